# Initial kernel scaffold; baseline (speedup 1.0000x reference)
import jax
import jax.numpy as jnp
from jax.experimental import pallas as pl


def kernel(ei_campaign_hosted_on_platform, ei_platform_rev_hosted_on_campaign, ei_campaign_uses_channel, ei_channel_rev_uses_campaign, ei_platform_supports_channel, ei_campaign_uses_creative, ei_creative_rev_uses_campaign, ei_creative_designed_with_template, ei_campaign_associated_with_keywords, ei_keywords_rev_associated_with_campaign, ei_campaign_managed_by_network, ei_platform_optimized_for_keywords, ei_campaign_belongs_to_advertiser, ei_campaign_targeted_with_search_tag, ei_search_tag_rev_targeted_with_campaign, x_campaign, x_platform, x_channel, x_creative, x_keywords, x_search_tag, x_advertiser, x_template, x_network, p0_0_wl, p0_0_wr, p0_0_b, p0_1_wl, p0_1_wr, p0_1_b, p0_2_wl, p0_2_wr, p0_2_b, p0_3_wl, p0_3_wr, p0_3_b, p0_4_wl, p0_4_wr, p0_4_b, p0_5_wl, p0_5_wr, p0_5_b, p0_6_wl, p0_6_wr, p0_6_b, p0_7_wl, p0_7_wr, p0_7_b, p0_8_wl, p0_8_wr, p0_8_b, p0_9_wl, p0_9_wr, p0_9_b, p0_10_wl, p0_10_wr, p0_10_b, p0_11_wl, p0_11_wr, p0_11_b, p0_12_wl, p0_12_wr, p0_12_b, p0_13_wl, p0_13_wr, p0_13_b, p0_14_wl, p0_14_wr, p0_14_b, ln0_g, ln0_b, p1_0_wl, p1_0_wr, p1_0_b, p1_1_wl, p1_1_wr, p1_1_b, p1_2_wl, p1_2_wr, p1_2_b, p1_3_wl, p1_3_wr, p1_3_b, p1_4_wl, p1_4_wr, p1_4_b, p1_5_wl, p1_5_wr, p1_5_b, p1_6_wl, p1_6_wr, p1_6_b, p1_7_wl, p1_7_wr, p1_7_b, p1_8_wl, p1_8_wr, p1_8_b, p1_9_wl, p1_9_wr, p1_9_b, p1_10_wl, p1_10_wr, p1_10_b, p1_11_wl, p1_11_wr, p1_11_b, p1_12_wl, p1_12_wr, p1_12_b, p1_13_wl, p1_13_wr, p1_13_b, p1_14_wl, p1_14_wr, p1_14_b, ln1_g, ln1_b, p2_0_wl, p2_0_wr, p2_0_b, p2_1_wl, p2_1_wr, p2_1_b, p2_2_wl, p2_2_wr, p2_2_b, p2_3_wl, p2_3_wr, p2_3_b, p2_4_wl, p2_4_wr, p2_4_b, p2_5_wl, p2_5_wr, p2_5_b, p2_6_wl, p2_6_wr, p2_6_b, p2_7_wl, p2_7_wr, p2_7_b, p2_8_wl, p2_8_wr, p2_8_b, p2_9_wl, p2_9_wr, p2_9_b, p2_10_wl, p2_10_wr, p2_10_b, p2_11_wl, p2_11_wr, p2_11_b, p2_12_wl, p2_12_wr, p2_12_b, p2_13_wl, p2_13_wr, p2_13_b, p2_14_wl, p2_14_wr, p2_14_b, ln2_g, ln2_b, fc_w, fc_b):
    raise NotImplementedError("write your pallas kernel here")


# speedup vs baseline: 1.9546x; 1.9546x over previous
import functools

import jax
import jax.numpy as jnp
from jax import lax
from jax.experimental import pallas as pl
from jax.experimental.pallas import tpu as pltpu
from jax.experimental.pallas import tpu_sc as plsc

N = 10000
D = 128
E = 40000
RP = 10112
RPT = RP // 16
EP = 40960
EPT = EP // 16
CH = 128
NCH = EPT // CH
NS = 15
NT = 9
ROWB = 512
NBLK = (N + ROWB - 1) // ROWB
NBLKP = (RP + ROWB - 1) // ROWB

SLOT2REL = [1, 3, 6, 9, 14, 2, 4, 8, 11, 0, 5, 7, 10, 12, 13]
SRC_T = [1, 2, 3, 4, 5, 0, 1, 0, 1, 0, 0, 3, 0, 0, 0]
K_OF_T = [5, 1, 2, 1, 2, 1, 1, 1, 1]
DST_T = [0, 0, 0, 0, 0, 2, 2, 4, 4, 1, 3, 7, 8, 6, 5]
_seen = {}
POS_OF_SLOT = []
for _s in range(NS):
    _t = DST_T[_s]
    _j = _seen.get(_t, 0)
    _seen[_t] = _j + 1
    POS_OF_SLOT.append(5 * _t + _j)
CORE_OF = [0] * 8 + [1] * 7


def _build_y(x_stack, wl_stack):
    def src_map(s):
        v = 0
        for i, t in enumerate(SRC_T):
            if t:
                v = jnp.where(s == i, t, v)
        return v

    def body(x_ref, wl_ref, y_ref):
        y_ref[0] = jnp.dot(x_ref[0], wl_ref[0],
                           preferred_element_type=jnp.float32)

    return pl.pallas_call(
        body,
        grid=(NS, NBLK),
        in_specs=[
            pl.BlockSpec((1, ROWB, D), lambda s, nb: (src_map(s), nb, 0)),
            pl.BlockSpec((1, D, D), lambda s, nb: (s, 0, 0)),
        ],
        out_specs=pl.BlockSpec((1, ROWB, D), lambda s, nb: (s, nb, 0)),
        out_shape=jax.ShapeDtypeStruct((NS, N, D), jnp.float32),
    )(x_stack, wl_stack)


@functools.lru_cache(maxsize=1)
def _make_sc_count():
    mesh = plsc.VectorSubcoreMesh(core_axis_name="c", subcore_axis_name="s")

    @functools.partial(
        pl.kernel,
        out_type=jax.ShapeDtypeStruct((NS, RP, D), jnp.float32),
        mesh=mesh,
        scratch_types=[
            pltpu.VMEM_SHARED((RP, D), jnp.float32),
            pltpu.VMEM((CH,), jnp.int32),
            pltpu.VMEM((CH, D), jnp.float32),
            pltpu.SemaphoreType.DMA,
        ],
    )
    def sc_count(dsts_hbm, zeros_hbm, ones_hbm, cnt_hbm, acc, didx, obuf,
                 sem):
        cid = lax.axis_index("c")
        sid = lax.axis_index("s")
        rlo = sid * RPT
        pltpu.sync_copy(ones_hbm, obuf)
        for slot in range(NS):
            @pl.when(cid == CORE_OF[slot])
            def _(slot=slot):
                pltpu.sync_copy(zeros_hbm.at[pl.ds(rlo, RPT)],
                                acc.at[pl.ds(rlo, RPT)])
                plsc.subcore_barrier()
                ebase = slot * EP + sid * EPT

                def chunk(k):
                    pltpu.sync_copy(dsts_hbm.at[pl.ds(ebase + k * CH, CH)],
                                    didx)
                    pltpu.sync_copy(obuf, acc.at[didx], add=True)

                pl.loop(0, NCH)(chunk)
                plsc.subcore_barrier()
                pltpu.sync_copy(acc.at[pl.ds(rlo, RPT)],
                                cnt_hbm.at[slot, pl.ds(rlo, RPT)])

    return sc_count


@functools.lru_cache(maxsize=1)
def _make_sc_agg():
    mesh = plsc.VectorSubcoreMesh(core_axis_name="c", subcore_axis_name="s")

    @functools.partial(
        pl.kernel,
        out_type=jax.ShapeDtypeStruct((5 * NT, RP, D), jnp.float32),
        mesh=mesh,
        scratch_types=[
            pltpu.VMEM_SHARED((RP, D), jnp.float32),
            pltpu.VMEM((CH,), jnp.int32),
            pltpu.VMEM((CH,), jnp.int32),
            pltpu.VMEM((CH, D), jnp.float32),
            pltpu.SemaphoreType.DMA,
        ],
    )
    def sc_agg(y_hbm, srcs_hbm, dsts_hbm, zeros_hbm, s_hbm, acc, sidx, didx,
               gbuf, sem):
        cid = lax.axis_index("c")
        sid = lax.axis_index("s")
        rlo = sid * RPT
        for slot in range(NS):
            @pl.when(cid == CORE_OF[slot])
            def _(slot=slot):
                pltpu.sync_copy(zeros_hbm.at[pl.ds(rlo, RPT)],
                                acc.at[pl.ds(rlo, RPT)])
                plsc.subcore_barrier()
                ebase = slot * EP + sid * EPT

                def chunk(k):
                    off = ebase + k * CH
                    pltpu.sync_copy(srcs_hbm.at[pl.ds(off, CH)], sidx)
                    pltpu.sync_copy(dsts_hbm.at[pl.ds(off, CH)], didx)
                    pltpu.async_copy(y_hbm.at[sidx], gbuf, sem).wait()
                    pltpu.sync_copy(gbuf, acc.at[didx], add=True)

                pl.loop(0, NCH)(chunk)
                plsc.subcore_barrier()
                pltpu.sync_copy(acc.at[pl.ds(rlo, RPT)],
                                s_hbm.at[POS_OF_SLOT[slot], pl.ds(rlo, RPT)])

    return sc_agg


def _compress_counts(cnt_full):
    def body(c_ref, o_ref):
        cols = [[jnp.zeros((ROWB,), jnp.float32)] * 5 for _ in range(NT)]
        for slot in range(NS):
            c = c_ref[slot][:, 0]
            p = POS_OF_SLOT[slot]
            cols[p // 5][p % 5] = 1.0 / jnp.maximum(c, 1.0)
        o_ref[...] = jnp.stack([jnp.stack(cs, axis=1) for cs in cols])

    return pl.pallas_call(
        body,
        grid=(NBLKP,),
        in_specs=[pl.BlockSpec((NS, ROWB, D), lambda nb: (0, nb, 0))],
        out_specs=pl.BlockSpec((NT, ROWB, 5), lambda nb: (0, nb, 0)),
        out_shape=jax.ShapeDtypeStruct((NT, RP, 5), jnp.float32),
    )(cnt_full)


def _combine(s_buf, inv_cnt, x_stack, wr_pad, b_pad, ln_g, ln_b):
    def body(s_ref, ic_ref, x_ref, wr_ref, b_ref, g_ref, bb_ref, o_ref):
        t = pl.program_id(0)
        k = jnp.where(t == 0, 5, jnp.where((t == 2) | (t == 4), 2, 1))
        acc = jnp.zeros((ROWB, D), jnp.float32)
        wr_sum = jnp.zeros((D, D), jnp.float32)
        b_sum = jnp.zeros((D,), jnp.float32)
        for j in range(5):
            mean = s_ref[j] * ic_ref[0, :, j][:, None]
            m = j < k
            acc = acc + jnp.where(m, mean, 0.0)
            wr_sum = wr_sum + jnp.where(m, wr_ref[j], 0.0)
            b_sum = b_sum + jnp.where(m, b_ref[0, j], 0.0)
        h = acc + jnp.dot(x_ref[0], wr_sum,
                          preferred_element_type=jnp.float32)
        h = (h + b_sum[None, :]) / k.astype(jnp.float32)
        h = jnp.where(h > 0, h, 0.01 * h)
        mu = jnp.mean(h, axis=1, keepdims=True)
        var = jnp.mean((h - mu) ** 2, axis=1, keepdims=True)
        o_ref[0] = (h - mu) * lax.rsqrt(var + 1e-5) * g_ref[0][None, :] \
            + bb_ref[0][None, :]

    return pl.pallas_call(
        body,
        grid=(NT, NBLK),
        in_specs=[
            pl.BlockSpec((5, ROWB, D), lambda t, nb: (t, nb, 0)),
            pl.BlockSpec((1, ROWB, 5), lambda t, nb: (t, nb, 0)),
            pl.BlockSpec((1, ROWB, D), lambda t, nb: (t, nb, 0)),
            pl.BlockSpec((5, D, D), lambda t, nb: (t, 0, 0)),
            pl.BlockSpec((1, 5, D), lambda t, nb: (t, 0, 0)),
            pl.BlockSpec((1, D), lambda t, nb: (0, 0)),
            pl.BlockSpec((1, D), lambda t, nb: (0, 0)),
        ],
        out_specs=pl.BlockSpec((1, ROWB, D), lambda t, nb: (t, nb, 0)),
        out_shape=jax.ShapeDtypeStruct((NT, N, D), jnp.float32),
    )(s_buf, inv_cnt, x_stack, wr_pad, b_pad, ln_g, ln_b)


def _final(x_stack, w, b):
    def body(x_ref, w_ref, b_ref, o_ref):
        acc = jnp.zeros((ROWB, 1), jnp.float32)
        for t in range(NT):
            acc = acc + jnp.sum(x_ref[t] * w_ref[t][None, :], axis=1,
                                keepdims=True)
        o_ref[...] = jax.nn.sigmoid(acc + b_ref[0, 0])

    return pl.pallas_call(
        body,
        grid=(NBLK,),
        in_specs=[
            pl.BlockSpec((NT, ROWB, D), lambda nb: (0, nb, 0)),
            pl.BlockSpec((NT, D), lambda nb: (0, 0)),
            pl.BlockSpec((1, 1), lambda nb: (0, 0)),
        ],
        out_specs=pl.BlockSpec((ROWB, 1), lambda nb: (nb, 0)),
        out_shape=jax.ShapeDtypeStruct((N, 1), jnp.float32),
    )(x_stack, w, b)


def kernel(ei_campaign_hosted_on_platform, ei_platform_rev_hosted_on_campaign, ei_campaign_uses_channel, ei_channel_rev_uses_campaign, ei_platform_supports_channel, ei_campaign_uses_creative, ei_creative_rev_uses_campaign, ei_creative_designed_with_template, ei_campaign_associated_with_keywords, ei_keywords_rev_associated_with_campaign, ei_campaign_managed_by_network, ei_platform_optimized_for_keywords, ei_campaign_belongs_to_advertiser, ei_campaign_targeted_with_search_tag, ei_search_tag_rev_targeted_with_campaign, x_campaign, x_platform, x_channel, x_creative, x_keywords, x_search_tag, x_advertiser, x_template, x_network, p0_0_wl, p0_0_wr, p0_0_b, p0_1_wl, p0_1_wr, p0_1_b, p0_2_wl, p0_2_wr, p0_2_b, p0_3_wl, p0_3_wr, p0_3_b, p0_4_wl, p0_4_wr, p0_4_b, p0_5_wl, p0_5_wr, p0_5_b, p0_6_wl, p0_6_wr, p0_6_b, p0_7_wl, p0_7_wr, p0_7_b, p0_8_wl, p0_8_wr, p0_8_b, p0_9_wl, p0_9_wr, p0_9_b, p0_10_wl, p0_10_wr, p0_10_b, p0_11_wl, p0_11_wr, p0_11_b, p0_12_wl, p0_12_wr, p0_12_b, p0_13_wl, p0_13_wr, p0_13_b, p0_14_wl, p0_14_wr, p0_14_b, ln0_g, ln0_b, p1_0_wl, p1_0_wr, p1_0_b, p1_1_wl, p1_1_wr, p1_1_b, p1_2_wl, p1_2_wr, p1_2_b, p1_3_wl, p1_3_wr, p1_3_b, p1_4_wl, p1_4_wr, p1_4_b, p1_5_wl, p1_5_wr, p1_5_b, p1_6_wl, p1_6_wr, p1_6_b, p1_7_wl, p1_7_wr, p1_7_b, p1_8_wl, p1_8_wr, p1_8_b, p1_9_wl, p1_9_wr, p1_9_b, p1_10_wl, p1_10_wr, p1_10_b, p1_11_wl, p1_11_wr, p1_11_b, p1_12_wl, p1_12_wr, p1_12_b, p1_13_wl, p1_13_wr, p1_13_b, p1_14_wl, p1_14_wr, p1_14_b, ln1_g, ln1_b, p2_0_wl, p2_0_wr, p2_0_b, p2_1_wl, p2_1_wr, p2_1_b, p2_2_wl, p2_2_wr, p2_2_b, p2_3_wl, p2_3_wr, p2_3_b, p2_4_wl, p2_4_wr, p2_4_b, p2_5_wl, p2_5_wr, p2_5_b, p2_6_wl, p2_6_wr, p2_6_b, p2_7_wl, p2_7_wr, p2_7_b, p2_8_wl, p2_8_wr, p2_8_b, p2_9_wl, p2_9_wr, p2_9_b, p2_10_wl, p2_10_wr, p2_10_b, p2_11_wl, p2_11_wr, p2_11_b, p2_12_wl, p2_12_wr, p2_12_b, p2_13_wl, p2_13_wr, p2_13_b, p2_14_wl, p2_14_wr, p2_14_b, ln2_g, ln2_b, fc_w, fc_b):
    kw = dict(locals())
    eis = [
        ei_campaign_hosted_on_platform, ei_platform_rev_hosted_on_campaign,
        ei_campaign_uses_channel, ei_channel_rev_uses_campaign,
        ei_platform_supports_channel, ei_campaign_uses_creative,
        ei_creative_rev_uses_campaign, ei_creative_designed_with_template,
        ei_campaign_associated_with_keywords,
        ei_keywords_rev_associated_with_campaign,
        ei_campaign_managed_by_network, ei_platform_optimized_for_keywords,
        ei_campaign_belongs_to_advertiser, ei_campaign_targeted_with_search_tag,
        ei_search_tag_rev_targeted_with_campaign,
    ]
    x_stack = jnp.stack([x_campaign, x_platform, x_channel, x_creative,
                         x_keywords, x_search_tag, x_advertiser, x_template,
                         x_network])

    srcs, dsts = [], []
    for slot, rel in enumerate(SLOT2REL):
        ei = eis[rel]
        s_ = jnp.concatenate([ei[0], jnp.zeros((EP - E,), jnp.int32)])
        d_ = jnp.concatenate([ei[1], jnp.full((EP - E,), N, jnp.int32)])
        srcs.append(s_ + slot * N)
        dsts.append(d_)
    srcs = jnp.concatenate(srcs)
    dsts = jnp.concatenate(dsts)
    zeros_rp = jnp.zeros((RP, D), jnp.float32)
    ones_ch = jnp.ones((CH, D), jnp.float32)

    cnt_full = _make_sc_count()(dsts, zeros_rp, ones_ch)
    inv_cnt = _compress_counts(cnt_full)

    zero_w = jnp.zeros((D, D), jnp.float32)
    zero_b = jnp.zeros((D,), jnp.float32)
    for L in range(3):
        wl_stack = jnp.stack([kw["p%d_%d_wl" % (L, r)] for r in SLOT2REL])
        wr45, b45 = [zero_w] * 45, [zero_b] * 45
        for slot, rel in enumerate(SLOT2REL):
            wr45[POS_OF_SLOT[slot]] = kw["p%d_%d_wr" % (L, rel)]
            b45[POS_OF_SLOT[slot]] = kw["p%d_%d_b" % (L, rel)]
        wr_pad = jnp.stack(wr45)
        b_pad = jnp.stack(b45).reshape(NT, 5, D)

        y = _build_y(x_stack, wl_stack).reshape(NS * N, D)
        s_buf = _make_sc_agg()(y, srcs, dsts, zeros_rp)
        x_stack = _combine(s_buf, inv_cnt, x_stack, wr_pad, b_pad,
                           kw["ln%d_g" % L].reshape(1, D),
                           kw["ln%d_b" % L].reshape(1, D))

    out = _final(x_stack, fc_w[:, 0].reshape(NT, D), fc_b.reshape(1, 1))
    return out[:, 0]


# speedup vs baseline: 2.3919x; 1.2237x over previous
import functools

import jax
import jax.numpy as jnp
from jax import lax
from jax.experimental import pallas as pl
from jax.experimental.pallas import tpu as pltpu
from jax.experimental.pallas import tpu_sc as plsc

N = 10000
D = 128
E = 40000
RP = 10112
RPT = RP // 16
EP = 40960
EPT = EP // 16
CH = 128
NCH = EPT // CH
NS = 15
NT = 9
ROWB = 512
NBLK = (N + ROWB - 1) // ROWB
NBLKP = (RP + ROWB - 1) // ROWB

SLOT2REL = [1, 3, 6, 9, 14, 2, 4, 8, 11, 0, 5, 7, 10, 12, 13]
SRC_T = [1, 2, 3, 4, 5, 0, 1, 0, 1, 0, 0, 3, 0, 0, 0]
K_OF_T = [5, 1, 2, 1, 2, 1, 1, 1, 1]
DST_T = [0, 0, 0, 0, 0, 2, 2, 4, 4, 1, 3, 7, 8, 6, 5]
_seen = {}
POS_OF_SLOT = []
for _s in range(NS):
    _t = DST_T[_s]
    _j = _seen.get(_t, 0)
    _seen[_t] = _j + 1
    POS_OF_SLOT.append(5 * _t + _j)
CORE_OF = [0] * 8 + [1] * 7


def _build_y(x_stack, wl_stack):
    def src_map(s):
        v = 0
        for i, t in enumerate(SRC_T):
            if t:
                v = jnp.where(s == i, t, v)
        return v

    def body(x_ref, wl_ref, y_ref):
        y_ref[0] = jnp.dot(x_ref[0], wl_ref[0],
                           preferred_element_type=jnp.float32)

    return pl.pallas_call(
        body,
        grid=(NS, NBLK),
        in_specs=[
            pl.BlockSpec((1, ROWB, D), lambda s, nb: (src_map(s), nb, 0)),
            pl.BlockSpec((1, D, D), lambda s, nb: (s, 0, 0)),
        ],
        out_specs=pl.BlockSpec((1, ROWB, D), lambda s, nb: (s, nb, 0)),
        out_shape=jax.ShapeDtypeStruct((NS, N, D), jnp.float32),
    )(x_stack, wl_stack)


@functools.lru_cache(maxsize=1)
def _make_sc_count():
    mesh = plsc.VectorSubcoreMesh(core_axis_name="c", subcore_axis_name="s")

    @functools.partial(
        pl.kernel,
        out_type=jax.ShapeDtypeStruct((NS, RP, D), jnp.float32),
        mesh=mesh,
        scratch_types=[
            pltpu.VMEM_SHARED((RP, D), jnp.float32),
            pltpu.VMEM((CH,), jnp.int32),
            pltpu.VMEM((CH,), jnp.int32),
            pltpu.VMEM((CH, D), jnp.float32),
            pltpu.SemaphoreType.DMA,
            pltpu.SemaphoreType.DMA,
        ],
    )
    def sc_count(dsts_hbm, zeros_hbm, ones_hbm, cnt_hbm, acc, db0, db1, obuf,
                 semd0, semd1):
        dbufs = (db0, db1)
        semds = (semd0, semd1)
        cid = lax.axis_index("c")
        sid = lax.axis_index("s")
        rlo = sid * RPT
        pltpu.sync_copy(ones_hbm, obuf)
        for slot in range(NS):
            @pl.when(cid == CORE_OF[slot])
            def _(slot=slot):
                pltpu.sync_copy(zeros_hbm.at[pl.ds(rlo, RPT)],
                                acc.at[pl.ds(rlo, RPT)])
                plsc.subcore_barrier()
                ebase = slot * EP + sid * EPT
                for b in range(2):
                    pltpu.async_copy(
                        dsts_hbm.at[pl.ds(ebase + b * CH, CH)],
                        dbufs[b], semds[b])

                def step(kk):
                    for b in range(2):
                        k = kk + b
                        pltpu.make_async_copy(
                            dsts_hbm.at[pl.ds(0, CH)], dbufs[b],
                            semds[b]).wait()
                        pltpu.sync_copy(obuf, acc.at[dbufs[b]], add=True)

                        @pl.when(k + 2 < NCH)
                        def _(k=k, b=b):
                            pltpu.async_copy(
                                dsts_hbm.at[pl.ds(ebase + (k + 2) * CH, CH)],
                                dbufs[b], semds[b])

                pl.loop(0, NCH, step=2)(step)
                plsc.subcore_barrier()
                pltpu.sync_copy(acc.at[pl.ds(rlo, RPT)],
                                cnt_hbm.at[slot, pl.ds(rlo, RPT)])

    return sc_count


@functools.lru_cache(maxsize=1)
def _make_sc_agg():
    mesh = plsc.VectorSubcoreMesh(core_axis_name="c", subcore_axis_name="s")

    @functools.partial(
        pl.kernel,
        out_type=jax.ShapeDtypeStruct((5 * NT, RP, D), jnp.float32),
        mesh=mesh,
        scratch_types=[
            pltpu.VMEM_SHARED((RP, D), jnp.float32),
            pltpu.VMEM((EPT,), jnp.int32),
            pltpu.VMEM((CH,), jnp.int32),
            pltpu.VMEM((CH,), jnp.int32),
            pltpu.VMEM((CH, D), jnp.float32),
            pltpu.VMEM((CH, D), jnp.float32),
            pltpu.SemaphoreType.DMA,
            pltpu.SemaphoreType.DMA,
            pltpu.SemaphoreType.DMA,
            pltpu.SemaphoreType.DMA,
        ],
    )
    def sc_agg(y_hbm, srcs_hbm, dsts_hbm, zeros_hbm, s_hbm, acc, sidx_all,
               db0, db1, gb0, gb1, semd0, semd1, semg0, semg1):
        dbufs = (db0, db1)
        gbufs = (gb0, gb1)
        semds = (semd0, semd1)
        semgs = (semg0, semg1)
        cid = lax.axis_index("c")
        sid = lax.axis_index("s")
        rlo = sid * RPT
        for slot in range(NS):
            @pl.when(cid == CORE_OF[slot])
            def _(slot=slot):
                pltpu.sync_copy(zeros_hbm.at[pl.ds(rlo, RPT)],
                                acc.at[pl.ds(rlo, RPT)])
                plsc.subcore_barrier()
                ebase = slot * EP + sid * EPT
                pltpu.sync_copy(srcs_hbm.at[pl.ds(ebase, EPT)], sidx_all)
                for b in range(2):
                    pltpu.async_copy(
                        dsts_hbm.at[pl.ds(ebase + b * CH, CH)],
                        dbufs[b], semds[b])
                    pltpu.async_copy(
                        y_hbm.at[sidx_all.at[pl.ds(b * CH, CH)]],
                        gbufs[b], semgs[b])

                def step(kk):
                    for b in range(2):
                        k = kk + b
                        pltpu.make_async_copy(
                            dsts_hbm.at[pl.ds(0, CH)], dbufs[b],
                            semds[b]).wait()
                        pltpu.make_async_copy(
                            y_hbm.at[pl.ds(0, CH)], gbufs[b],
                            semgs[b]).wait()
                        pltpu.sync_copy(gbufs[b], acc.at[dbufs[b]],
                                        add=True)

                        @pl.when(k + 2 < NCH)
                        def _(k=k, b=b):
                            pltpu.async_copy(
                                dsts_hbm.at[pl.ds(ebase + (k + 2) * CH, CH)],
                                dbufs[b], semds[b])
                            pltpu.async_copy(
                                y_hbm.at[sidx_all.at[pl.ds((k + 2) * CH, CH)]],
                                gbufs[b], semgs[b])

                pl.loop(0, NCH, step=2)(step)
                plsc.subcore_barrier()
                pltpu.sync_copy(acc.at[pl.ds(rlo, RPT)],
                                s_hbm.at[POS_OF_SLOT[slot], pl.ds(rlo, RPT)])

    return sc_agg


def _compress_counts(cnt_full):
    def body(c_ref, o_ref):
        cols = [[jnp.zeros((ROWB,), jnp.float32)] * 5 for _ in range(NT)]
        for slot in range(NS):
            c = c_ref[slot][:, 0]
            p = POS_OF_SLOT[slot]
            cols[p // 5][p % 5] = 1.0 / jnp.maximum(c, 1.0)
        o_ref[...] = jnp.stack([jnp.stack(cs, axis=1) for cs in cols])

    return pl.pallas_call(
        body,
        grid=(NBLKP,),
        in_specs=[pl.BlockSpec((NS, ROWB, D), lambda nb: (0, nb, 0))],
        out_specs=pl.BlockSpec((NT, ROWB, 5), lambda nb: (0, nb, 0)),
        out_shape=jax.ShapeDtypeStruct((NT, RP, 5), jnp.float32),
    )(cnt_full)


def _combine(s_buf, inv_cnt, x_stack, wr_pad, b_pad, ln_g, ln_b):
    def body(s_ref, ic_ref, x_ref, wr_ref, b_ref, g_ref, bb_ref, o_ref):
        t = pl.program_id(0)
        k = jnp.where(t == 0, 5, jnp.where((t == 2) | (t == 4), 2, 1))
        acc = jnp.zeros((ROWB, D), jnp.float32)
        wr_sum = jnp.zeros((D, D), jnp.float32)
        b_sum = jnp.zeros((D,), jnp.float32)
        for j in range(5):
            mean = s_ref[j] * ic_ref[0, :, j][:, None]
            m = j < k
            acc = acc + jnp.where(m, mean, 0.0)
            wr_sum = wr_sum + jnp.where(m, wr_ref[j], 0.0)
            b_sum = b_sum + jnp.where(m, b_ref[0, j], 0.0)
        h = acc + jnp.dot(x_ref[0], wr_sum,
                          preferred_element_type=jnp.float32)
        h = (h + b_sum[None, :]) / k.astype(jnp.float32)
        h = jnp.where(h > 0, h, 0.01 * h)
        mu = jnp.mean(h, axis=1, keepdims=True)
        var = jnp.mean((h - mu) ** 2, axis=1, keepdims=True)
        o_ref[0] = (h - mu) * lax.rsqrt(var + 1e-5) * g_ref[0][None, :] \
            + bb_ref[0][None, :]

    return pl.pallas_call(
        body,
        grid=(NT, NBLK),
        in_specs=[
            pl.BlockSpec((5, ROWB, D), lambda t, nb: (t, nb, 0)),
            pl.BlockSpec((1, ROWB, 5), lambda t, nb: (t, nb, 0)),
            pl.BlockSpec((1, ROWB, D), lambda t, nb: (t, nb, 0)),
            pl.BlockSpec((5, D, D), lambda t, nb: (t, 0, 0)),
            pl.BlockSpec((1, 5, D), lambda t, nb: (t, 0, 0)),
            pl.BlockSpec((1, D), lambda t, nb: (0, 0)),
            pl.BlockSpec((1, D), lambda t, nb: (0, 0)),
        ],
        out_specs=pl.BlockSpec((1, ROWB, D), lambda t, nb: (t, nb, 0)),
        out_shape=jax.ShapeDtypeStruct((NT, N, D), jnp.float32),
    )(s_buf, inv_cnt, x_stack, wr_pad, b_pad, ln_g, ln_b)


def _final(x_stack, w, b):
    def body(x_ref, w_ref, b_ref, o_ref):
        acc = jnp.zeros((ROWB, 1), jnp.float32)
        for t in range(NT):
            acc = acc + jnp.sum(x_ref[t] * w_ref[t][None, :], axis=1,
                                keepdims=True)
        o_ref[...] = jax.nn.sigmoid(acc + b_ref[0, 0])

    return pl.pallas_call(
        body,
        grid=(NBLK,),
        in_specs=[
            pl.BlockSpec((NT, ROWB, D), lambda nb: (0, nb, 0)),
            pl.BlockSpec((NT, D), lambda nb: (0, 0)),
            pl.BlockSpec((1, 1), lambda nb: (0, 0)),
        ],
        out_specs=pl.BlockSpec((ROWB, 1), lambda nb: (nb, 0)),
        out_shape=jax.ShapeDtypeStruct((N, 1), jnp.float32),
    )(x_stack, w, b)


def kernel(ei_campaign_hosted_on_platform, ei_platform_rev_hosted_on_campaign, ei_campaign_uses_channel, ei_channel_rev_uses_campaign, ei_platform_supports_channel, ei_campaign_uses_creative, ei_creative_rev_uses_campaign, ei_creative_designed_with_template, ei_campaign_associated_with_keywords, ei_keywords_rev_associated_with_campaign, ei_campaign_managed_by_network, ei_platform_optimized_for_keywords, ei_campaign_belongs_to_advertiser, ei_campaign_targeted_with_search_tag, ei_search_tag_rev_targeted_with_campaign, x_campaign, x_platform, x_channel, x_creative, x_keywords, x_search_tag, x_advertiser, x_template, x_network, p0_0_wl, p0_0_wr, p0_0_b, p0_1_wl, p0_1_wr, p0_1_b, p0_2_wl, p0_2_wr, p0_2_b, p0_3_wl, p0_3_wr, p0_3_b, p0_4_wl, p0_4_wr, p0_4_b, p0_5_wl, p0_5_wr, p0_5_b, p0_6_wl, p0_6_wr, p0_6_b, p0_7_wl, p0_7_wr, p0_7_b, p0_8_wl, p0_8_wr, p0_8_b, p0_9_wl, p0_9_wr, p0_9_b, p0_10_wl, p0_10_wr, p0_10_b, p0_11_wl, p0_11_wr, p0_11_b, p0_12_wl, p0_12_wr, p0_12_b, p0_13_wl, p0_13_wr, p0_13_b, p0_14_wl, p0_14_wr, p0_14_b, ln0_g, ln0_b, p1_0_wl, p1_0_wr, p1_0_b, p1_1_wl, p1_1_wr, p1_1_b, p1_2_wl, p1_2_wr, p1_2_b, p1_3_wl, p1_3_wr, p1_3_b, p1_4_wl, p1_4_wr, p1_4_b, p1_5_wl, p1_5_wr, p1_5_b, p1_6_wl, p1_6_wr, p1_6_b, p1_7_wl, p1_7_wr, p1_7_b, p1_8_wl, p1_8_wr, p1_8_b, p1_9_wl, p1_9_wr, p1_9_b, p1_10_wl, p1_10_wr, p1_10_b, p1_11_wl, p1_11_wr, p1_11_b, p1_12_wl, p1_12_wr, p1_12_b, p1_13_wl, p1_13_wr, p1_13_b, p1_14_wl, p1_14_wr, p1_14_b, ln1_g, ln1_b, p2_0_wl, p2_0_wr, p2_0_b, p2_1_wl, p2_1_wr, p2_1_b, p2_2_wl, p2_2_wr, p2_2_b, p2_3_wl, p2_3_wr, p2_3_b, p2_4_wl, p2_4_wr, p2_4_b, p2_5_wl, p2_5_wr, p2_5_b, p2_6_wl, p2_6_wr, p2_6_b, p2_7_wl, p2_7_wr, p2_7_b, p2_8_wl, p2_8_wr, p2_8_b, p2_9_wl, p2_9_wr, p2_9_b, p2_10_wl, p2_10_wr, p2_10_b, p2_11_wl, p2_11_wr, p2_11_b, p2_12_wl, p2_12_wr, p2_12_b, p2_13_wl, p2_13_wr, p2_13_b, p2_14_wl, p2_14_wr, p2_14_b, ln2_g, ln2_b, fc_w, fc_b):
    kw = dict(locals())
    eis = [
        ei_campaign_hosted_on_platform, ei_platform_rev_hosted_on_campaign,
        ei_campaign_uses_channel, ei_channel_rev_uses_campaign,
        ei_platform_supports_channel, ei_campaign_uses_creative,
        ei_creative_rev_uses_campaign, ei_creative_designed_with_template,
        ei_campaign_associated_with_keywords,
        ei_keywords_rev_associated_with_campaign,
        ei_campaign_managed_by_network, ei_platform_optimized_for_keywords,
        ei_campaign_belongs_to_advertiser, ei_campaign_targeted_with_search_tag,
        ei_search_tag_rev_targeted_with_campaign,
    ]
    x_stack = jnp.stack([x_campaign, x_platform, x_channel, x_creative,
                         x_keywords, x_search_tag, x_advertiser, x_template,
                         x_network])

    srcs, dsts = [], []
    for slot, rel in enumerate(SLOT2REL):
        ei = eis[rel]
        s_ = jnp.concatenate([ei[0], jnp.zeros((EP - E,), jnp.int32)])
        d_ = jnp.concatenate([ei[1], jnp.full((EP - E,), N, jnp.int32)])
        srcs.append(s_ + slot * N)
        dsts.append(d_)
    srcs = jnp.concatenate(srcs)
    dsts = jnp.concatenate(dsts)
    zeros_rp = jnp.zeros((RP, D), jnp.float32)
    ones_ch = jnp.ones((CH, D), jnp.float32)

    cnt_full = _make_sc_count()(dsts, zeros_rp, ones_ch)
    inv_cnt = _compress_counts(cnt_full)

    zero_w = jnp.zeros((D, D), jnp.float32)
    zero_b = jnp.zeros((D,), jnp.float32)
    for L in range(3):
        wl_stack = jnp.stack([kw["p%d_%d_wl" % (L, r)] for r in SLOT2REL])
        wr45, b45 = [zero_w] * 45, [zero_b] * 45
        for slot, rel in enumerate(SLOT2REL):
            wr45[POS_OF_SLOT[slot]] = kw["p%d_%d_wr" % (L, rel)]
            b45[POS_OF_SLOT[slot]] = kw["p%d_%d_b" % (L, rel)]
        wr_pad = jnp.stack(wr45)
        b_pad = jnp.stack(b45).reshape(NT, 5, D)

        y = _build_y(x_stack, wl_stack).reshape(NS * N, D)
        s_buf = _make_sc_agg()(y, srcs, dsts, zeros_rp)
        x_stack = _combine(s_buf, inv_cnt, x_stack, wr_pad, b_pad,
                           kw["ln%d_g" % L].reshape(1, D),
                           kw["ln%d_b" % L].reshape(1, D))

    out = _final(x_stack, fc_w[:, 0].reshape(NT, D), fc_b.reshape(1, 1))
    return out[:, 0]


# speedup vs baseline: 2.4134x; 1.0090x over previous
import functools

import jax
import jax.numpy as jnp
from jax import lax
from jax.experimental import pallas as pl
from jax.experimental.pallas import tpu as pltpu
from jax.experimental.pallas import tpu_sc as plsc

N = 10000
D = 128
E = 40000
RP = 10112
RPT = RP // 16
EP = 40960
EPT = EP // 16
CH = 64
NCH = EPT // CH
NS = 15
NT = 9
ROWB = 512
NBLK = (N + ROWB - 1) // ROWB
NBLKP = (RP + ROWB - 1) // ROWB

SLOT2REL = [1, 3, 6, 9, 14, 2, 4, 8, 11, 0, 5, 7, 10, 12, 13]
SRC_T = [1, 2, 3, 4, 5, 0, 1, 0, 1, 0, 0, 3, 0, 0, 0]
K_OF_T = [5, 1, 2, 1, 2, 1, 1, 1, 1]
DST_T = [0, 0, 0, 0, 0, 2, 2, 4, 4, 1, 3, 7, 8, 6, 5]
_seen = {}
POS_OF_SLOT = []
for _s in range(NS):
    _t = DST_T[_s]
    _j = _seen.get(_t, 0)
    _seen[_t] = _j + 1
    POS_OF_SLOT.append(5 * _t + _j)
CORE_OF = [0] * 8 + [1] * 7


def _build_y(x_stack, wl_stack):
    def src_map(s):
        v = 0
        for i, t in enumerate(SRC_T):
            if t:
                v = jnp.where(s == i, t, v)
        return v

    def body(x_ref, wl_ref, y_ref):
        y_ref[0] = jnp.dot(x_ref[0], wl_ref[0],
                           preferred_element_type=jnp.float32)

    return pl.pallas_call(
        body,
        grid=(NS, NBLK),
        in_specs=[
            pl.BlockSpec((1, ROWB, D), lambda s, nb: (src_map(s), nb, 0)),
            pl.BlockSpec((1, D, D), lambda s, nb: (s, 0, 0)),
        ],
        out_specs=pl.BlockSpec((1, ROWB, D), lambda s, nb: (s, nb, 0)),
        out_shape=jax.ShapeDtypeStruct((NS, N, D), jnp.float32),
    )(x_stack, wl_stack)


@functools.lru_cache(maxsize=1)
def _make_sc_count():
    mesh = plsc.VectorSubcoreMesh(core_axis_name="c", subcore_axis_name="s")

    @functools.partial(
        pl.kernel,
        out_type=jax.ShapeDtypeStruct((NS, RP, D), jnp.float32),
        mesh=mesh,
        scratch_types=[
            pltpu.VMEM_SHARED((RP, D), jnp.float32),
            pltpu.VMEM((CH,), jnp.int32),
            pltpu.VMEM((CH,), jnp.int32),
            pltpu.VMEM((CH, D), jnp.float32),
            pltpu.SemaphoreType.DMA,
            pltpu.SemaphoreType.DMA,
        ],
    )
    def sc_count(dsts_hbm, zeros_hbm, ones_hbm, cnt_hbm, acc, db0, db1, obuf,
                 semd0, semd1):
        dbufs = (db0, db1)
        semds = (semd0, semd1)
        cid = lax.axis_index("c")
        sid = lax.axis_index("s")
        rlo = sid * RPT
        pltpu.sync_copy(ones_hbm, obuf)
        for slot in range(NS):
            @pl.when(cid == CORE_OF[slot])
            def _(slot=slot):
                pltpu.sync_copy(zeros_hbm.at[pl.ds(rlo, RPT)],
                                acc.at[pl.ds(rlo, RPT)])
                plsc.subcore_barrier()
                ebase = slot * EP + sid * EPT
                for b in range(2):
                    pltpu.async_copy(
                        dsts_hbm.at[pl.ds(ebase + b * CH, CH)],
                        dbufs[b], semds[b])

                def step(kk):
                    for b in range(2):
                        k = kk + b
                        pltpu.make_async_copy(
                            dsts_hbm.at[pl.ds(0, CH)], dbufs[b],
                            semds[b]).wait()
                        pltpu.sync_copy(obuf, acc.at[dbufs[b]], add=True)

                        @pl.when(k + 2 < NCH)
                        def _(k=k, b=b):
                            pltpu.async_copy(
                                dsts_hbm.at[pl.ds(ebase + (k + 2) * CH, CH)],
                                dbufs[b], semds[b])

                pl.loop(0, NCH, step=2)(step)
                plsc.subcore_barrier()
                pltpu.sync_copy(acc.at[pl.ds(rlo, RPT)],
                                cnt_hbm.at[slot, pl.ds(rlo, RPT)])

    return sc_count


@functools.lru_cache(maxsize=1)
def _make_sc_agg():
    mesh = plsc.VectorSubcoreMesh(core_axis_name="c", subcore_axis_name="s")

    @functools.partial(
        pl.kernel,
        out_type=jax.ShapeDtypeStruct((5 * NT, RP, D), jnp.float32),
        mesh=mesh,
        scratch_types=(
            [pltpu.VMEM_SHARED((RP, D), jnp.float32),
             pltpu.VMEM((EPT,), jnp.int32)]
            + [pltpu.VMEM((CH,), jnp.int32) for _ in range(4)]
            + [pltpu.VMEM((CH, D), jnp.float32) for _ in range(4)]
            + [pltpu.SemaphoreType.DMA for _ in range(8)]
        ),
    )
    def sc_agg(y_hbm, srcs_hbm, dsts_hbm, zeros_hbm, s_hbm, acc, sidx_all,
               *bufs):
        dbufs = bufs[0:4]
        gbufs = bufs[4:8]
        semds = bufs[8:12]
        semgs = bufs[12:16]
        NB = 4
        cid = lax.axis_index("c")
        sid = lax.axis_index("s")
        rlo = sid * RPT
        for slot in range(NS):
            @pl.when(cid == CORE_OF[slot])
            def _(slot=slot):
                pltpu.sync_copy(zeros_hbm.at[pl.ds(rlo, RPT)],
                                acc.at[pl.ds(rlo, RPT)])
                plsc.subcore_barrier()
                ebase = slot * EP + sid * EPT
                pltpu.sync_copy(srcs_hbm.at[pl.ds(ebase, EPT)], sidx_all)
                for b in range(NB):
                    pltpu.async_copy(
                        dsts_hbm.at[pl.ds(ebase + b * CH, CH)],
                        dbufs[b], semds[b])
                    pltpu.async_copy(
                        y_hbm.at[sidx_all.at[pl.ds(b * CH, CH)]],
                        gbufs[b], semgs[b])

                def step(kk):
                    for b in range(NB):
                        k = kk + b
                        pltpu.make_async_copy(
                            dsts_hbm.at[pl.ds(0, CH)], dbufs[b],
                            semds[b]).wait()
                        pltpu.make_async_copy(
                            y_hbm.at[pl.ds(0, CH)], gbufs[b],
                            semgs[b]).wait()
                        pltpu.sync_copy(gbufs[b], acc.at[dbufs[b]],
                                        add=True)

                        @pl.when(k + NB < NCH)
                        def _(k=k, b=b):
                            pltpu.async_copy(
                                dsts_hbm.at[pl.ds(ebase + (k + NB) * CH, CH)],
                                dbufs[b], semds[b])
                            pltpu.async_copy(
                                y_hbm.at[sidx_all.at[pl.ds((k + NB) * CH, CH)]],
                                gbufs[b], semgs[b])

                pl.loop(0, NCH, step=NB)(step)
                plsc.subcore_barrier()
                pltpu.sync_copy(acc.at[pl.ds(rlo, RPT)],
                                s_hbm.at[POS_OF_SLOT[slot], pl.ds(rlo, RPT)])

    return sc_agg


def _compress_counts(cnt_full):
    def body(c_ref, o_ref):
        cols = [[jnp.zeros((ROWB,), jnp.float32)] * 5 for _ in range(NT)]
        for slot in range(NS):
            c = c_ref[slot][:, 0]
            p = POS_OF_SLOT[slot]
            cols[p // 5][p % 5] = 1.0 / jnp.maximum(c, 1.0)
        o_ref[...] = jnp.stack([jnp.stack(cs, axis=1) for cs in cols])

    return pl.pallas_call(
        body,
        grid=(NBLKP,),
        in_specs=[pl.BlockSpec((NS, ROWB, D), lambda nb: (0, nb, 0))],
        out_specs=pl.BlockSpec((NT, ROWB, 5), lambda nb: (0, nb, 0)),
        out_shape=jax.ShapeDtypeStruct((NT, RP, 5), jnp.float32),
    )(cnt_full)


def _combine(s_buf, inv_cnt, x_stack, wr_pad, b_pad, ln_g, ln_b):
    def body(s_ref, ic_ref, x_ref, wr_ref, b_ref, g_ref, bb_ref, o_ref):
        t = pl.program_id(0)
        k = jnp.where(t == 0, 5, jnp.where((t == 2) | (t == 4), 2, 1))
        acc = jnp.zeros((ROWB, D), jnp.float32)
        wr_sum = jnp.zeros((D, D), jnp.float32)
        b_sum = jnp.zeros((D,), jnp.float32)
        for j in range(5):
            mean = s_ref[j] * ic_ref[0, :, j][:, None]
            m = j < k
            acc = acc + jnp.where(m, mean, 0.0)
            wr_sum = wr_sum + jnp.where(m, wr_ref[j], 0.0)
            b_sum = b_sum + jnp.where(m, b_ref[0, j], 0.0)
        h = acc + jnp.dot(x_ref[0], wr_sum,
                          preferred_element_type=jnp.float32)
        h = (h + b_sum[None, :]) / k.astype(jnp.float32)
        h = jnp.where(h > 0, h, 0.01 * h)
        mu = jnp.mean(h, axis=1, keepdims=True)
        var = jnp.mean((h - mu) ** 2, axis=1, keepdims=True)
        o_ref[0] = (h - mu) * lax.rsqrt(var + 1e-5) * g_ref[0][None, :] \
            + bb_ref[0][None, :]

    return pl.pallas_call(
        body,
        grid=(NT, NBLK),
        in_specs=[
            pl.BlockSpec((5, ROWB, D), lambda t, nb: (t, nb, 0)),
            pl.BlockSpec((1, ROWB, 5), lambda t, nb: (t, nb, 0)),
            pl.BlockSpec((1, ROWB, D), lambda t, nb: (t, nb, 0)),
            pl.BlockSpec((5, D, D), lambda t, nb: (t, 0, 0)),
            pl.BlockSpec((1, 5, D), lambda t, nb: (t, 0, 0)),
            pl.BlockSpec((1, D), lambda t, nb: (0, 0)),
            pl.BlockSpec((1, D), lambda t, nb: (0, 0)),
        ],
        out_specs=pl.BlockSpec((1, ROWB, D), lambda t, nb: (t, nb, 0)),
        out_shape=jax.ShapeDtypeStruct((NT, N, D), jnp.float32),
    )(s_buf, inv_cnt, x_stack, wr_pad, b_pad, ln_g, ln_b)


def _final(x_stack, w, b):
    def body(x_ref, w_ref, b_ref, o_ref):
        acc = jnp.zeros((ROWB, 1), jnp.float32)
        for t in range(NT):
            acc = acc + jnp.sum(x_ref[t] * w_ref[t][None, :], axis=1,
                                keepdims=True)
        o_ref[...] = jax.nn.sigmoid(acc + b_ref[0, 0])

    return pl.pallas_call(
        body,
        grid=(NBLK,),
        in_specs=[
            pl.BlockSpec((NT, ROWB, D), lambda nb: (0, nb, 0)),
            pl.BlockSpec((NT, D), lambda nb: (0, 0)),
            pl.BlockSpec((1, 1), lambda nb: (0, 0)),
        ],
        out_specs=pl.BlockSpec((ROWB, 1), lambda nb: (nb, 0)),
        out_shape=jax.ShapeDtypeStruct((N, 1), jnp.float32),
    )(x_stack, w, b)


def kernel(ei_campaign_hosted_on_platform, ei_platform_rev_hosted_on_campaign, ei_campaign_uses_channel, ei_channel_rev_uses_campaign, ei_platform_supports_channel, ei_campaign_uses_creative, ei_creative_rev_uses_campaign, ei_creative_designed_with_template, ei_campaign_associated_with_keywords, ei_keywords_rev_associated_with_campaign, ei_campaign_managed_by_network, ei_platform_optimized_for_keywords, ei_campaign_belongs_to_advertiser, ei_campaign_targeted_with_search_tag, ei_search_tag_rev_targeted_with_campaign, x_campaign, x_platform, x_channel, x_creative, x_keywords, x_search_tag, x_advertiser, x_template, x_network, p0_0_wl, p0_0_wr, p0_0_b, p0_1_wl, p0_1_wr, p0_1_b, p0_2_wl, p0_2_wr, p0_2_b, p0_3_wl, p0_3_wr, p0_3_b, p0_4_wl, p0_4_wr, p0_4_b, p0_5_wl, p0_5_wr, p0_5_b, p0_6_wl, p0_6_wr, p0_6_b, p0_7_wl, p0_7_wr, p0_7_b, p0_8_wl, p0_8_wr, p0_8_b, p0_9_wl, p0_9_wr, p0_9_b, p0_10_wl, p0_10_wr, p0_10_b, p0_11_wl, p0_11_wr, p0_11_b, p0_12_wl, p0_12_wr, p0_12_b, p0_13_wl, p0_13_wr, p0_13_b, p0_14_wl, p0_14_wr, p0_14_b, ln0_g, ln0_b, p1_0_wl, p1_0_wr, p1_0_b, p1_1_wl, p1_1_wr, p1_1_b, p1_2_wl, p1_2_wr, p1_2_b, p1_3_wl, p1_3_wr, p1_3_b, p1_4_wl, p1_4_wr, p1_4_b, p1_5_wl, p1_5_wr, p1_5_b, p1_6_wl, p1_6_wr, p1_6_b, p1_7_wl, p1_7_wr, p1_7_b, p1_8_wl, p1_8_wr, p1_8_b, p1_9_wl, p1_9_wr, p1_9_b, p1_10_wl, p1_10_wr, p1_10_b, p1_11_wl, p1_11_wr, p1_11_b, p1_12_wl, p1_12_wr, p1_12_b, p1_13_wl, p1_13_wr, p1_13_b, p1_14_wl, p1_14_wr, p1_14_b, ln1_g, ln1_b, p2_0_wl, p2_0_wr, p2_0_b, p2_1_wl, p2_1_wr, p2_1_b, p2_2_wl, p2_2_wr, p2_2_b, p2_3_wl, p2_3_wr, p2_3_b, p2_4_wl, p2_4_wr, p2_4_b, p2_5_wl, p2_5_wr, p2_5_b, p2_6_wl, p2_6_wr, p2_6_b, p2_7_wl, p2_7_wr, p2_7_b, p2_8_wl, p2_8_wr, p2_8_b, p2_9_wl, p2_9_wr, p2_9_b, p2_10_wl, p2_10_wr, p2_10_b, p2_11_wl, p2_11_wr, p2_11_b, p2_12_wl, p2_12_wr, p2_12_b, p2_13_wl, p2_13_wr, p2_13_b, p2_14_wl, p2_14_wr, p2_14_b, ln2_g, ln2_b, fc_w, fc_b):
    kw = dict(locals())
    eis = [
        ei_campaign_hosted_on_platform, ei_platform_rev_hosted_on_campaign,
        ei_campaign_uses_channel, ei_channel_rev_uses_campaign,
        ei_platform_supports_channel, ei_campaign_uses_creative,
        ei_creative_rev_uses_campaign, ei_creative_designed_with_template,
        ei_campaign_associated_with_keywords,
        ei_keywords_rev_associated_with_campaign,
        ei_campaign_managed_by_network, ei_platform_optimized_for_keywords,
        ei_campaign_belongs_to_advertiser, ei_campaign_targeted_with_search_tag,
        ei_search_tag_rev_targeted_with_campaign,
    ]
    x_stack = jnp.stack([x_campaign, x_platform, x_channel, x_creative,
                         x_keywords, x_search_tag, x_advertiser, x_template,
                         x_network])

    srcs, dsts = [], []
    for slot, rel in enumerate(SLOT2REL):
        ei = eis[rel]
        s_ = jnp.concatenate([ei[0], jnp.zeros((EP - E,), jnp.int32)])
        d_ = jnp.concatenate([ei[1], jnp.full((EP - E,), N, jnp.int32)])
        srcs.append(s_ + slot * N)
        dsts.append(d_)
    srcs = jnp.concatenate(srcs)
    dsts = jnp.concatenate(dsts)
    zeros_rp = jnp.zeros((RP, D), jnp.float32)
    ones_ch = jnp.ones((CH, D), jnp.float32)

    cnt_full = _make_sc_count()(dsts, zeros_rp, ones_ch)
    inv_cnt = _compress_counts(cnt_full)

    zero_w = jnp.zeros((D, D), jnp.float32)
    zero_b = jnp.zeros((D,), jnp.float32)
    for L in range(3):
        wl_stack = jnp.stack([kw["p%d_%d_wl" % (L, r)] for r in SLOT2REL])
        wr45, b45 = [zero_w] * 45, [zero_b] * 45
        for slot, rel in enumerate(SLOT2REL):
            wr45[POS_OF_SLOT[slot]] = kw["p%d_%d_wr" % (L, rel)]
            b45[POS_OF_SLOT[slot]] = kw["p%d_%d_b" % (L, rel)]
        wr_pad = jnp.stack(wr45)
        b_pad = jnp.stack(b45).reshape(NT, 5, D)

        y = _build_y(x_stack, wl_stack).reshape(NS * N, D)
        s_buf = _make_sc_agg()(y, srcs, dsts, zeros_rp)
        x_stack = _combine(s_buf, inv_cnt, x_stack, wr_pad, b_pad,
                           kw["ln%d_g" % L].reshape(1, D),
                           kw["ln%d_b" % L].reshape(1, D))

    out = _final(x_stack, fc_w[:, 0].reshape(NT, D), fc_b.reshape(1, 1))
    return out[:, 0]


# speedup vs baseline: 2.5595x; 1.0605x over previous
import functools

import jax
import jax.numpy as jnp
from jax import lax
from jax.experimental import pallas as pl
from jax.experimental.pallas import tpu as pltpu
from jax.experimental.pallas import tpu_sc as plsc

N = 10000
D = 128
E = 40000
RP = 10112
RPT = RP // 16
EP = 40960
EPT = EP // 16
CH = 64
NCH = EPT // CH
NB = 4
NS = 15
NT = 9
ROWB = 512
NBLK = (N + ROWB - 1) // ROWB
NBLKP = (RP + ROWB - 1) // ROWB

SLOT2REL = [1, 3, 6, 9, 14, 2, 4, 8, 11, 0, 5, 7, 10, 12, 13]
SRC_T = [1, 2, 3, 4, 5, 0, 1, 0, 1, 0, 0, 3, 0, 0, 0]
DST_T = [0, 0, 0, 0, 0, 2, 2, 4, 4, 1, 3, 7, 8, 6, 5]
SING_TYPES = [1, 3, 7, 8, 6, 5]
_seen = {}
POS_OF_SLOT = []
for _s in range(NS):
    _t = DST_T[_s]
    _j = _seen.get(_t, 0)
    _seen[_t] = _j + 1
    POS_OF_SLOT.append(5 * _t + _j)
NSA, NSB = 7, 8
CORE_A = (0, 0, 0, 0, 1, 1, 1)
CORE_B = (0, 0, 0, 0, 1, 1, 1, 1)


def _build_y(x_stack, wl_stack, src_t):
    n = len(src_t)

    def src_map(s):
        v = 0
        for i, t in enumerate(src_t):
            if t:
                v = jnp.where(s == i, t, v)
        return v

    def body(x_ref, wl_ref, y_ref):
        y_ref[0] = jnp.dot(x_ref[0], wl_ref[0],
                           preferred_element_type=jnp.float32)

    return pl.pallas_call(
        body,
        grid=(n, NBLK),
        in_specs=[
            pl.BlockSpec((1, ROWB, D), lambda s, nb: (src_map(s), nb, 0)),
            pl.BlockSpec((1, D, D), lambda s, nb: (s, 0, 0)),
        ],
        out_specs=pl.BlockSpec((1, ROWB, D), lambda s, nb: (s, nb, 0)),
        out_shape=jax.ShapeDtypeStruct((n, N, D), jnp.float32),
    )(x_stack, wl_stack)


@functools.lru_cache(maxsize=1)
def _make_sc_count():
    mesh = plsc.VectorSubcoreMesh(core_axis_name="c", subcore_axis_name="s")
    core_of = [0] * 8 + [1] * 7

    @functools.partial(
        pl.kernel,
        out_type=jax.ShapeDtypeStruct((NS, RP, D), jnp.float32),
        mesh=mesh,
        scratch_types=[
            pltpu.VMEM_SHARED((RP, D), jnp.float32),
            pltpu.VMEM((CH,), jnp.int32),
            pltpu.VMEM((CH,), jnp.int32),
            pltpu.VMEM((CH, D), jnp.float32),
            pltpu.SemaphoreType.DMA,
            pltpu.SemaphoreType.DMA,
        ],
    )
    def sc_count(dsts_hbm, zeros_hbm, ones_hbm, cnt_hbm, acc, db0, db1, obuf,
                 semd0, semd1):
        dbufs = (db0, db1)
        semds = (semd0, semd1)
        cid = lax.axis_index("c")
        sid = lax.axis_index("s")
        rlo = sid * RPT
        pltpu.sync_copy(ones_hbm, obuf)
        for slot in range(NS):
            @pl.when(cid == core_of[slot])
            def _(slot=slot):
                pltpu.sync_copy(zeros_hbm.at[pl.ds(rlo, RPT)],
                                acc.at[pl.ds(rlo, RPT)])
                plsc.subcore_barrier()
                ebase = slot * EP + sid * EPT
                for b in range(2):
                    pltpu.async_copy(
                        dsts_hbm.at[pl.ds(ebase + b * CH, CH)],
                        dbufs[b], semds[b])

                def step(kk):
                    for b in range(2):
                        k = kk + b
                        pltpu.make_async_copy(
                            dsts_hbm.at[pl.ds(0, CH)], dbufs[b],
                            semds[b]).wait()
                        pltpu.sync_copy(obuf, acc.at[dbufs[b]], add=True)

                        @pl.when(k + 2 < NCH)
                        def _(k=k, b=b):
                            pltpu.async_copy(
                                dsts_hbm.at[pl.ds(ebase + (k + 2) * CH, CH)],
                                dbufs[b], semds[b])

                pl.loop(0, NCH, step=2)(step)
                plsc.subcore_barrier()
                pltpu.sync_copy(acc.at[pl.ds(rlo, RPT)],
                                cnt_hbm.at[slot, pl.ds(rlo, RPT)])

    return sc_count


@functools.lru_cache(maxsize=4)
def _make_sc_agg(core_of, out_splits):
    mesh = plsc.VectorSubcoreMesh(core_axis_name="c", subcore_axis_name="s")
    nsl = len(core_of)
    out_type = [jax.ShapeDtypeStruct((g, RP, D), jnp.float32)
                for g in out_splits]
    slot_pos = []
    for gi, g in enumerate(out_splits):
        for j in range(g):
            slot_pos.append((gi, j))

    @functools.partial(
        pl.kernel,
        out_type=out_type,
        mesh=mesh,
        scratch_types=(
            [pltpu.VMEM_SHARED((RP, D), jnp.float32),
             pltpu.VMEM((EPT,), jnp.int32)]
            + [pltpu.VMEM((CH,), jnp.int32) for _ in range(NB)]
            + [pltpu.VMEM((CH, D), jnp.float32) for _ in range(NB)]
            + [pltpu.SemaphoreType.DMA for _ in range(2 * NB)]
        ),
    )
    def sc_agg(y_hbm, srcs_hbm, dsts_hbm, zeros_hbm, *rest):
        outs = rest[:len(out_splits)]
        rest = rest[len(out_splits):]
        acc, sidx_all = rest[0], rest[1]
        dbufs = rest[2:2 + NB]
        gbufs = rest[2 + NB:2 + 2 * NB]
        semds = rest[2 + 2 * NB:2 + 3 * NB]
        semgs = rest[2 + 3 * NB:2 + 4 * NB]
        cid = lax.axis_index("c")
        sid = lax.axis_index("s")
        rlo = sid * RPT
        for slot in range(nsl):
            @pl.when(cid == core_of[slot])
            def _(slot=slot):
                pltpu.sync_copy(zeros_hbm.at[pl.ds(rlo, RPT)],
                                acc.at[pl.ds(rlo, RPT)])
                plsc.subcore_barrier()
                ebase = slot * EP + sid * EPT
                pltpu.sync_copy(srcs_hbm.at[pl.ds(ebase, EPT)], sidx_all)
                for b in range(NB):
                    pltpu.async_copy(
                        dsts_hbm.at[pl.ds(ebase + b * CH, CH)],
                        dbufs[b], semds[b])
                    pltpu.async_copy(
                        y_hbm.at[sidx_all.at[pl.ds(b * CH, CH)]],
                        gbufs[b], semgs[b])

                def step(kk):
                    for b in range(NB):
                        k = kk + b
                        pltpu.make_async_copy(
                            dsts_hbm.at[pl.ds(0, CH)], dbufs[b],
                            semds[b]).wait()
                        pltpu.make_async_copy(
                            y_hbm.at[pl.ds(0, CH)], gbufs[b],
                            semgs[b]).wait()
                        pltpu.sync_copy(gbufs[b], acc.at[dbufs[b]],
                                        add=True)

                        @pl.when(k + NB < NCH)
                        def _(k=k, b=b):
                            pltpu.async_copy(
                                dsts_hbm.at[pl.ds(ebase + (k + NB) * CH, CH)],
                                dbufs[b], semds[b])
                            pltpu.async_copy(
                                y_hbm.at[sidx_all.at[pl.ds((k + NB) * CH, CH)]],
                                gbufs[b], semgs[b])

                pl.loop(0, NCH, step=NB)(step)
                plsc.subcore_barrier()
                gi, j = slot_pos[slot]
                pltpu.sync_copy(acc.at[pl.ds(rlo, RPT)],
                                outs[gi].at[j, pl.ds(rlo, RPT)])

    return sc_agg


def _compress_counts(cnt_full):
    def body(c_ref, o_camp, o_chan, o_keyw, o_sing):
        inv = [1.0 / jnp.maximum(c_ref[s][:, 0], 1.0) for s in range(NS)]
        o_camp[...] = jnp.stack(inv[0:5])[None]
        o_chan[...] = jnp.stack(inv[5:7])[None]
        o_keyw[...] = jnp.stack(inv[7:9])[None]
        o_sing[...] = jnp.stack(inv[9:15])[:, None, :]

    return pl.pallas_call(
        body,
        grid=(NBLKP,),
        in_specs=[pl.BlockSpec((NS, ROWB, D), lambda nb: (0, nb, 0))],
        out_specs=[
            pl.BlockSpec((1, 5, ROWB), lambda nb: (0, 0, nb)),
            pl.BlockSpec((1, 2, ROWB), lambda nb: (0, 0, nb)),
            pl.BlockSpec((1, 2, ROWB), lambda nb: (0, 0, nb)),
            pl.BlockSpec((6, 1, ROWB), lambda nb: (0, 0, nb)),
        ],
        out_shape=[
            jax.ShapeDtypeStruct((1, 5, RP), jnp.float32),
            jax.ShapeDtypeStruct((1, 2, RP), jnp.float32),
            jax.ShapeDtypeStruct((1, 2, RP), jnp.float32),
            jax.ShapeDtypeStruct((6, 1, RP), jnp.float32),
        ],
    )(cnt_full)


def _g_of_t(t):
    v = 0
    for g, ty in enumerate(SING_TYPES):
        if g:
            v = jnp.where(t == ty, g, v)
    return v


def _combine(s_camp, s_chan, s_keyw, s_sing, inv_camp, inv_chan, inv_keyw,
             inv_sing, x_stack, wr_pad, b_pad, ln_g, ln_b):
    def body(sc_ref, sh_ref, sk_ref, ss_ref, icc_ref, ich_ref, ick_ref,
             ics_ref, x_ref, wr_ref, b_ref, g_ref, bb_ref, o_ref):
        t = pl.program_id(0)
        k = jnp.where(t == 0, 5, jnp.where((t == 2) | (t == 4), 2, 1))
        camp_acc = jnp.zeros((ROWB, D), jnp.float32)
        for j in range(5):
            camp_acc = camp_acc + sc_ref[j] * icc_ref[0, j][:, None]
        chan_acc = (sh_ref[0] * ich_ref[0, 0][:, None]
                    + sh_ref[1] * ich_ref[0, 1][:, None])
        keyw_acc = (sk_ref[0] * ick_ref[0, 0][:, None]
                    + sk_ref[1] * ick_ref[0, 1][:, None])
        sing_acc = ss_ref[0] * ics_ref[0, 0][:, None]
        is_sing = (t == 1) | (t == 3) | (t >= 5)
        acc = (jnp.where(t == 0, camp_acc, 0.0)
               + jnp.where(t == 2, chan_acc, 0.0)
               + jnp.where(t == 4, keyw_acc, 0.0)
               + jnp.where(is_sing, sing_acc, 0.0))
        wr_sum = jnp.zeros((D, D), jnp.float32)
        b_sum = jnp.zeros((D,), jnp.float32)
        for j in range(5):
            m = j < k
            wr_sum = wr_sum + jnp.where(m, wr_ref[j], 0.0)
            b_sum = b_sum + jnp.where(m, b_ref[0, j], 0.0)
        h = acc + jnp.dot(x_ref[0], wr_sum,
                          preferred_element_type=jnp.float32)
        h = (h + b_sum[None, :]) / k.astype(jnp.float32)
        h = jnp.where(h > 0, h, 0.01 * h)
        mu = jnp.mean(h, axis=1, keepdims=True)
        var = jnp.mean((h - mu) ** 2, axis=1, keepdims=True)
        o_ref[0] = (h - mu) * lax.rsqrt(var + 1e-5) * g_ref[0][None, :] \
            + bb_ref[0][None, :]

    sing = lambda t: (t == 1) | (t == 3) | (t >= 5)
    return pl.pallas_call(
        body,
        grid=(NT, NBLK),
        in_specs=[
            pl.BlockSpec((5, ROWB, D),
                         lambda t, nb: (0, jnp.where(t == 0, nb, 0), 0)),
            pl.BlockSpec((2, ROWB, D),
                         lambda t, nb: (0, jnp.where(t == 2, nb, 0), 0)),
            pl.BlockSpec((2, ROWB, D),
                         lambda t, nb: (0, jnp.where(t == 4, nb, 0), 0)),
            pl.BlockSpec((1, ROWB, D),
                         lambda t, nb: (_g_of_t(t),
                                        jnp.where(sing(t), nb, 0), 0)),
            pl.BlockSpec((1, 5, ROWB),
                         lambda t, nb: (0, 0, jnp.where(t == 0, nb, 0))),
            pl.BlockSpec((1, 2, ROWB),
                         lambda t, nb: (0, 0, jnp.where(t == 2, nb, 0))),
            pl.BlockSpec((1, 2, ROWB),
                         lambda t, nb: (0, 0, jnp.where(t == 4, nb, 0))),
            pl.BlockSpec((1, 1, ROWB),
                         lambda t, nb: (_g_of_t(t), 0,
                                        jnp.where(sing(t), nb, 0))),
            pl.BlockSpec((1, ROWB, D), lambda t, nb: (t, nb, 0)),
            pl.BlockSpec((5, D, D), lambda t, nb: (t, 0, 0)),
            pl.BlockSpec((1, 5, D), lambda t, nb: (t, 0, 0)),
            pl.BlockSpec((1, D), lambda t, nb: (0, 0)),
            pl.BlockSpec((1, D), lambda t, nb: (0, 0)),
        ],
        out_specs=pl.BlockSpec((1, ROWB, D), lambda t, nb: (t, nb, 0)),
        out_shape=jax.ShapeDtypeStruct((NT, N, D), jnp.float32),
    )(s_camp, s_chan, s_keyw, s_sing, inv_camp, inv_chan, inv_keyw,
      inv_sing, x_stack, wr_pad, b_pad, ln_g, ln_b)


def _final(x_stack, w, b):
    def body(x_ref, w_ref, b_ref, o_ref):
        acc = jnp.zeros((ROWB, 1), jnp.float32)
        for t in range(NT):
            acc = acc + jnp.sum(x_ref[t] * w_ref[t][None, :], axis=1,
                                keepdims=True)
        o_ref[...] = jax.nn.sigmoid(acc + b_ref[0, 0])

    return pl.pallas_call(
        body,
        grid=(NBLK,),
        in_specs=[
            pl.BlockSpec((NT, ROWB, D), lambda nb: (0, nb, 0)),
            pl.BlockSpec((NT, D), lambda nb: (0, 0)),
            pl.BlockSpec((1, 1), lambda nb: (0, 0)),
        ],
        out_specs=pl.BlockSpec((ROWB, 1), lambda nb: (nb, 0)),
        out_shape=jax.ShapeDtypeStruct((N, 1), jnp.float32),
    )(x_stack, w, b)


def kernel(ei_campaign_hosted_on_platform, ei_platform_rev_hosted_on_campaign, ei_campaign_uses_channel, ei_channel_rev_uses_campaign, ei_platform_supports_channel, ei_campaign_uses_creative, ei_creative_rev_uses_campaign, ei_creative_designed_with_template, ei_campaign_associated_with_keywords, ei_keywords_rev_associated_with_campaign, ei_campaign_managed_by_network, ei_platform_optimized_for_keywords, ei_campaign_belongs_to_advertiser, ei_campaign_targeted_with_search_tag, ei_search_tag_rev_targeted_with_campaign, x_campaign, x_platform, x_channel, x_creative, x_keywords, x_search_tag, x_advertiser, x_template, x_network, p0_0_wl, p0_0_wr, p0_0_b, p0_1_wl, p0_1_wr, p0_1_b, p0_2_wl, p0_2_wr, p0_2_b, p0_3_wl, p0_3_wr, p0_3_b, p0_4_wl, p0_4_wr, p0_4_b, p0_5_wl, p0_5_wr, p0_5_b, p0_6_wl, p0_6_wr, p0_6_b, p0_7_wl, p0_7_wr, p0_7_b, p0_8_wl, p0_8_wr, p0_8_b, p0_9_wl, p0_9_wr, p0_9_b, p0_10_wl, p0_10_wr, p0_10_b, p0_11_wl, p0_11_wr, p0_11_b, p0_12_wl, p0_12_wr, p0_12_b, p0_13_wl, p0_13_wr, p0_13_b, p0_14_wl, p0_14_wr, p0_14_b, ln0_g, ln0_b, p1_0_wl, p1_0_wr, p1_0_b, p1_1_wl, p1_1_wr, p1_1_b, p1_2_wl, p1_2_wr, p1_2_b, p1_3_wl, p1_3_wr, p1_3_b, p1_4_wl, p1_4_wr, p1_4_b, p1_5_wl, p1_5_wr, p1_5_b, p1_6_wl, p1_6_wr, p1_6_b, p1_7_wl, p1_7_wr, p1_7_b, p1_8_wl, p1_8_wr, p1_8_b, p1_9_wl, p1_9_wr, p1_9_b, p1_10_wl, p1_10_wr, p1_10_b, p1_11_wl, p1_11_wr, p1_11_b, p1_12_wl, p1_12_wr, p1_12_b, p1_13_wl, p1_13_wr, p1_13_b, p1_14_wl, p1_14_wr, p1_14_b, ln1_g, ln1_b, p2_0_wl, p2_0_wr, p2_0_b, p2_1_wl, p2_1_wr, p2_1_b, p2_2_wl, p2_2_wr, p2_2_b, p2_3_wl, p2_3_wr, p2_3_b, p2_4_wl, p2_4_wr, p2_4_b, p2_5_wl, p2_5_wr, p2_5_b, p2_6_wl, p2_6_wr, p2_6_b, p2_7_wl, p2_7_wr, p2_7_b, p2_8_wl, p2_8_wr, p2_8_b, p2_9_wl, p2_9_wr, p2_9_b, p2_10_wl, p2_10_wr, p2_10_b, p2_11_wl, p2_11_wr, p2_11_b, p2_12_wl, p2_12_wr, p2_12_b, p2_13_wl, p2_13_wr, p2_13_b, p2_14_wl, p2_14_wr, p2_14_b, ln2_g, ln2_b, fc_w, fc_b):
    kw = dict(locals())
    eis = [
        ei_campaign_hosted_on_platform, ei_platform_rev_hosted_on_campaign,
        ei_campaign_uses_channel, ei_channel_rev_uses_campaign,
        ei_platform_supports_channel, ei_campaign_uses_creative,
        ei_creative_rev_uses_campaign, ei_creative_designed_with_template,
        ei_campaign_associated_with_keywords,
        ei_keywords_rev_associated_with_campaign,
        ei_campaign_managed_by_network, ei_platform_optimized_for_keywords,
        ei_campaign_belongs_to_advertiser, ei_campaign_targeted_with_search_tag,
        ei_search_tag_rev_targeted_with_campaign,
    ]
    x_stack = jnp.stack([x_campaign, x_platform, x_channel, x_creative,
                         x_keywords, x_search_tag, x_advertiser, x_template,
                         x_network])

    srcs, dsts = [], []
    for slot, rel in enumerate(SLOT2REL):
        ei = eis[rel]
        s_ = jnp.concatenate([ei[0], jnp.zeros((EP - E,), jnp.int32)])
        d_ = jnp.concatenate([ei[1], jnp.full((EP - E,), N, jnp.int32)])
        half_slot = slot if slot < NSA else slot - NSA
        srcs.append(s_ + half_slot * N)
        dsts.append(d_)
    srcs_a = jnp.concatenate(srcs[:NSA])
    srcs_b = jnp.concatenate(srcs[NSA:])
    dsts_all = jnp.concatenate(dsts)
    dsts_a = jnp.concatenate(dsts[:NSA])
    dsts_b = jnp.concatenate(dsts[NSA:])
    zeros_rp = jnp.zeros((RP, D), jnp.float32)
    ones_ch = jnp.ones((CH, D), jnp.float32)

    cnt_full = _make_sc_count()(dsts_all, zeros_rp, ones_ch)
    inv_camp, inv_chan, inv_keyw, inv_sing = _compress_counts(cnt_full)

    agg_a = _make_sc_agg(CORE_A, (5, 2))
    agg_b = _make_sc_agg(CORE_B, (2, 6))

    zero_w = jnp.zeros((D, D), jnp.float32)
    zero_b = jnp.zeros((D,), jnp.float32)
    for L in range(3):
        wl = [kw["p%d_%d_wl" % (L, r)] for r in SLOT2REL]
        lng = kw["ln%d_g" % L].reshape(1, D)
        lnb = kw["ln%d_b" % L].reshape(1, D)
        wr45, b45 = [zero_w] * 45, [zero_b] * 45
        for slot, rel in enumerate(SLOT2REL):
            wr45[POS_OF_SLOT[slot]] = kw["p%d_%d_wr" % (L, rel)]
            b45[POS_OF_SLOT[slot]] = kw["p%d_%d_b" % (L, rel)]
        wr_pad = jnp.stack(wr45)
        b_pad = jnp.stack(b45).reshape(NT, 5, D)

        y_a = _build_y(x_stack, jnp.stack(wl[:NSA]), tuple(SRC_T[:NSA]))
        s_camp, s_chan = agg_a(y_a.reshape(NSA * N, D), srcs_a, dsts_a,
                               zeros_rp)
        y_b = _build_y(x_stack, jnp.stack(wl[NSA:]), tuple(SRC_T[NSA:]))
        s_keyw, s_sing = agg_b(y_b.reshape(NSB * N, D), srcs_b, dsts_b,
                               zeros_rp)

        x_stack = _combine(s_camp, s_chan, s_keyw, s_sing, inv_camp,
                           inv_chan, inv_keyw, inv_sing, x_stack, wr_pad,
                           b_pad, lng, lnb)

    out = _final(x_stack, fc_w[:, 0].reshape(NT, D), fc_b.reshape(1, 1))
    return out[:, 0]


# speedup vs baseline: 2.7884x; 1.0894x over previous
import functools

import jax
import jax.numpy as jnp
from jax import lax
from jax.experimental import pallas as pl
from jax.experimental.pallas import tpu as pltpu
from jax.experimental.pallas import tpu_sc as plsc

N = 10000
D = 128
E = 40000
RP = 10112
RPT = RP // 16
EP = 40960
EPT = EP // 16
CH = 64
NCH = EPT // CH
NB = 4
NS = 15
NT = 9
ROWB = 512
NBLK = (N + ROWB - 1) // ROWB
NBLKP = (RP + ROWB - 1) // ROWB

SLOT2REL = [1, 3, 6, 9, 14, 2, 4, 8, 11, 0, 5, 7, 10, 12, 13]
SRC_T = [1, 2, 3, 4, 5, 0, 1, 0, 1, 0, 0, 3, 0, 0, 0]
DST_T = [0, 0, 0, 0, 0, 2, 2, 4, 4, 1, 3, 7, 8, 6, 5]
SING_TYPES = [1, 3, 7, 8, 6, 5]
_seen = {}
POS_OF_SLOT = []
for _s in range(NS):
    _t = DST_T[_s]
    _j = _seen.get(_t, 0)
    _seen[_t] = _j + 1
    POS_OF_SLOT.append(5 * _t + _j)
NSA, NSB = 7, 8
CORE_A = (0, 0, 0, 0, 1, 1, 1)
CORE_B = (0, 0, 0, 0, 1, 1, 1, 1)


TB = [1, 3, 4, 5, 6, 7, 8]
A_IDX_OF_T = {0: 0, 2: 1}
B_IDX_OF_T = {t: i for i, t in enumerate(TB)}


def _build_y(xa, xb, wl_stack, src_t):
    n = len(src_t)
    use_a = [t in A_IDX_OF_T for t in src_t]

    def mk_map(idxs, use):
        def f(s):
            v = 0
            for i in range(n):
                if use[i] and idxs[i]:
                    v = jnp.where(s == i, idxs[i], v)
            return v

        def u(s):
            v = False
            for i in range(n):
                if use[i]:
                    v = jnp.logical_or(v, s == i)
            return v
        return f, u

    a_idx = [A_IDX_OF_T.get(t, 0) for t in src_t]
    b_idx = [B_IDX_OF_T.get(t, 0) for t in src_t]
    amap, a_on = mk_map(a_idx, use_a)
    bmap, b_on = mk_map(b_idx, [not u for u in use_a])

    def body(xa_ref, xb_ref, wl_ref, y_ref):
        s = pl.program_id(0)
        x = jnp.where(a_on(s), xa_ref[0], xb_ref[0])
        y_ref[0] = jnp.dot(x, wl_ref[0], preferred_element_type=jnp.float32)

    return pl.pallas_call(
        body,
        grid=(n, NBLK),
        in_specs=[
            pl.BlockSpec((1, ROWB, D),
                         lambda s, nb: (amap(s),
                                        jnp.where(a_on(s), nb, 0), 0)),
            pl.BlockSpec((1, ROWB, D),
                         lambda s, nb: (bmap(s),
                                        jnp.where(b_on(s), nb, 0), 0)),
            pl.BlockSpec((1, D, D), lambda s, nb: (s, 0, 0)),
        ],
        out_specs=pl.BlockSpec((1, ROWB, D), lambda s, nb: (s, nb, 0)),
        out_shape=jax.ShapeDtypeStruct((n, N, D), jnp.float32),
    )(xa, xb, wl_stack)


@functools.lru_cache(maxsize=1)
def _make_sc_count():
    mesh = plsc.VectorSubcoreMesh(core_axis_name="c", subcore_axis_name="s")
    core_of = [0] * 8 + [1] * 7

    @functools.partial(
        pl.kernel,
        out_type=jax.ShapeDtypeStruct((NS, RP, D), jnp.float32),
        mesh=mesh,
        scratch_types=[
            pltpu.VMEM_SHARED((RP, D), jnp.float32),
            pltpu.VMEM((CH,), jnp.int32),
            pltpu.VMEM((CH,), jnp.int32),
            pltpu.VMEM((CH, D), jnp.float32),
            pltpu.SemaphoreType.DMA,
            pltpu.SemaphoreType.DMA,
        ],
    )
    def sc_count(dsts_hbm, zeros_hbm, ones_hbm, cnt_hbm, acc, db0, db1, obuf,
                 semd0, semd1):
        dbufs = (db0, db1)
        semds = (semd0, semd1)
        cid = lax.axis_index("c")
        sid = lax.axis_index("s")
        rlo = sid * RPT
        pltpu.sync_copy(ones_hbm, obuf)
        for slot in range(NS):
            @pl.when(cid == core_of[slot])
            def _(slot=slot):
                pltpu.sync_copy(zeros_hbm.at[pl.ds(rlo, RPT)],
                                acc.at[pl.ds(rlo, RPT)])
                plsc.subcore_barrier()
                ebase = slot * EP + sid * EPT
                for b in range(2):
                    pltpu.async_copy(
                        dsts_hbm.at[pl.ds(ebase + b * CH, CH)],
                        dbufs[b], semds[b])

                def step(kk):
                    for b in range(2):
                        k = kk + b
                        pltpu.make_async_copy(
                            dsts_hbm.at[pl.ds(0, CH)], dbufs[b],
                            semds[b]).wait()
                        pltpu.sync_copy(obuf, acc.at[dbufs[b]], add=True)

                        @pl.when(k + 2 < NCH)
                        def _(k=k, b=b):
                            pltpu.async_copy(
                                dsts_hbm.at[pl.ds(ebase + (k + 2) * CH, CH)],
                                dbufs[b], semds[b])

                pl.loop(0, NCH, step=2)(step)
                plsc.subcore_barrier()
                pltpu.sync_copy(acc.at[pl.ds(rlo, RPT)],
                                cnt_hbm.at[slot, pl.ds(rlo, RPT)])

    return sc_count


@functools.lru_cache(maxsize=4)
def _make_sc_agg(core_of, out_splits):
    mesh = plsc.VectorSubcoreMesh(core_axis_name="c", subcore_axis_name="s")
    nsl = len(core_of)
    out_type = [jax.ShapeDtypeStruct((g, RP, D), jnp.float32)
                for g in out_splits]
    slot_pos = []
    for gi, g in enumerate(out_splits):
        for j in range(g):
            slot_pos.append((gi, j))

    @functools.partial(
        pl.kernel,
        out_type=out_type,
        mesh=mesh,
        scratch_types=(
            [pltpu.VMEM_SHARED((RP, D), jnp.float32),
             pltpu.VMEM((EPT,), jnp.int32)]
            + [pltpu.VMEM((CH,), jnp.int32) for _ in range(NB)]
            + [pltpu.VMEM((CH, D), jnp.float32) for _ in range(NB)]
            + [pltpu.SemaphoreType.DMA for _ in range(2 * NB)]
        ),
    )
    def sc_agg(y_hbm, srcs_hbm, dsts_hbm, zeros_hbm, *rest):
        outs = rest[:len(out_splits)]
        rest = rest[len(out_splits):]
        acc, sidx_all = rest[0], rest[1]
        dbufs = rest[2:2 + NB]
        gbufs = rest[2 + NB:2 + 2 * NB]
        semds = rest[2 + 2 * NB:2 + 3 * NB]
        semgs = rest[2 + 3 * NB:2 + 4 * NB]
        cid = lax.axis_index("c")
        sid = lax.axis_index("s")
        rlo = sid * RPT
        for slot in range(nsl):
            @pl.when(cid == core_of[slot])
            def _(slot=slot):
                pltpu.sync_copy(zeros_hbm.at[pl.ds(rlo, RPT)],
                                acc.at[pl.ds(rlo, RPT)])
                plsc.subcore_barrier()
                ebase = slot * EP + sid * EPT
                pltpu.sync_copy(srcs_hbm.at[pl.ds(ebase, EPT)], sidx_all)
                for b in range(NB):
                    pltpu.async_copy(
                        dsts_hbm.at[pl.ds(ebase + b * CH, CH)],
                        dbufs[b], semds[b])
                    pltpu.async_copy(
                        y_hbm.at[sidx_all.at[pl.ds(b * CH, CH)]],
                        gbufs[b], semgs[b])

                def step(kk):
                    for b in range(NB):
                        k = kk + b
                        pltpu.make_async_copy(
                            dsts_hbm.at[pl.ds(0, CH)], dbufs[b],
                            semds[b]).wait()
                        pltpu.make_async_copy(
                            y_hbm.at[pl.ds(0, CH)], gbufs[b],
                            semgs[b]).wait()
                        pltpu.sync_copy(gbufs[b], acc.at[dbufs[b]],
                                        add=True)

                        @pl.when(k + NB < NCH)
                        def _(k=k, b=b):
                            pltpu.async_copy(
                                dsts_hbm.at[pl.ds(ebase + (k + NB) * CH, CH)],
                                dbufs[b], semds[b])
                            pltpu.async_copy(
                                y_hbm.at[sidx_all.at[pl.ds((k + NB) * CH, CH)]],
                                gbufs[b], semgs[b])

                pl.loop(0, NCH, step=NB)(step)
                plsc.subcore_barrier()
                gi, j = slot_pos[slot]
                pltpu.sync_copy(acc.at[pl.ds(rlo, RPT)],
                                outs[gi].at[j, pl.ds(rlo, RPT)])

    return sc_agg


def _compress_counts(cnt_full):
    def body(c_ref, o_camp, o_chan, o_keyw, o_sing):
        inv = [1.0 / jnp.maximum(c_ref[s][:, 0], 1.0) for s in range(NS)]
        o_camp[...] = jnp.stack(inv[0:5])[None]
        o_chan[...] = jnp.stack(inv[5:7])[None]
        o_keyw[...] = jnp.stack(inv[7:9])[None]
        o_sing[...] = jnp.stack(inv[9:15])[:, None, :]

    return pl.pallas_call(
        body,
        grid=(NBLKP,),
        in_specs=[pl.BlockSpec((NS, ROWB, D), lambda nb: (0, nb, 0))],
        out_specs=[
            pl.BlockSpec((1, 5, ROWB), lambda nb: (0, 0, nb)),
            pl.BlockSpec((1, 2, ROWB), lambda nb: (0, 0, nb)),
            pl.BlockSpec((1, 2, ROWB), lambda nb: (0, 0, nb)),
            pl.BlockSpec((6, 1, ROWB), lambda nb: (0, 0, nb)),
        ],
        out_shape=[
            jax.ShapeDtypeStruct((1, 5, RP), jnp.float32),
            jax.ShapeDtypeStruct((1, 2, RP), jnp.float32),
            jax.ShapeDtypeStruct((1, 2, RP), jnp.float32),
            jax.ShapeDtypeStruct((6, 1, RP), jnp.float32),
        ],
    )(cnt_full)


SING_OF_G = [0, 1, 0, 5, 4, 2, 3]


def _ln_act(h, k, g_ref, bb_ref):
    h = h / k.astype(jnp.float32)
    h = jnp.where(h > 0, h, 0.01 * h)
    mu = jnp.mean(h, axis=1, keepdims=True)
    var = jnp.mean((h - mu) ** 2, axis=1, keepdims=True)
    return (h - mu) * lax.rsqrt(var + 1e-5) * g_ref[0][None, :] \
        + bb_ref[0][None, :]


def _combine_ks(s_keyw, s_sing, inv_keyw, inv_sing, xb, wr_b, b_b, ln_g,
                ln_b):
    def smap(g):
        v = 0
        for i, si in enumerate(SING_OF_G):
            if i != 2 and si:
                v = jnp.where(g == i, si, v)
        return v

    def body(sk_ref, ss_ref, ick_ref, ics_ref, x_ref, wr_ref, b_ref, g_ref,
             bb_ref, o_ref):
        g = pl.program_id(0)
        k = jnp.where(g == 2, 2, 1)
        keyw_acc = (sk_ref[0] * ick_ref[0, 0][:, None]
                    + sk_ref[1] * ick_ref[0, 1][:, None])
        sing_acc = ss_ref[0] * ics_ref[0, 0][:, None]
        acc = jnp.where(g == 2, keyw_acc, sing_acc)
        wr_sum = wr_ref[0] + wr_ref[1]
        b_sum = b_ref[0, 0] + b_ref[0, 1]
        h = acc + jnp.dot(x_ref[0], wr_sum,
                          preferred_element_type=jnp.float32) + b_sum[None, :]
        o_ref[0] = _ln_act(h, k, g_ref, bb_ref)

    return pl.pallas_call(
        body,
        grid=(7, NBLK),
        in_specs=[
            pl.BlockSpec((2, ROWB, D),
                         lambda g, nb: (0, jnp.where(g == 2, nb, 0), 0)),
            pl.BlockSpec((1, ROWB, D),
                         lambda g, nb: (smap(g),
                                        jnp.where(g == 2, 0, nb), 0)),
            pl.BlockSpec((1, 2, ROWB),
                         lambda g, nb: (0, 0, jnp.where(g == 2, nb, 0))),
            pl.BlockSpec((1, 1, ROWB),
                         lambda g, nb: (smap(g), 0,
                                        jnp.where(g == 2, 0, nb))),
            pl.BlockSpec((1, ROWB, D), lambda g, nb: (g, nb, 0)),
            pl.BlockSpec((2, D, D), lambda g, nb: (g, 0, 0)),
            pl.BlockSpec((1, 2, D), lambda g, nb: (g, 0, 0)),
            pl.BlockSpec((1, D), lambda g, nb: (0, 0)),
            pl.BlockSpec((1, D), lambda g, nb: (0, 0)),
        ],
        out_specs=pl.BlockSpec((1, ROWB, D), lambda g, nb: (g, nb, 0)),
        out_shape=jax.ShapeDtypeStruct((7, N, D), jnp.float32),
    )(s_keyw, s_sing, inv_keyw, inv_sing, xb, wr_b, b_b, ln_g, ln_b)


def _combine_cc(s_camp, s_chan, inv_camp, inv_chan, xa, wr_a, b_a, ln_g,
                ln_b):
    def body(sc_ref, sh_ref, icc_ref, ich_ref, x_ref, wr_ref, b_ref, g_ref,
             bb_ref, o_ref):
        g = pl.program_id(0)
        k = jnp.where(g == 0, 5, 2)
        camp_acc = jnp.zeros((ROWB, D), jnp.float32)
        for j in range(5):
            camp_acc = camp_acc + sc_ref[j] * icc_ref[0, j][:, None]
        chan_acc = (sh_ref[0] * ich_ref[0, 0][:, None]
                    + sh_ref[1] * ich_ref[0, 1][:, None])
        acc = jnp.where(g == 0, camp_acc, chan_acc)
        wr_sum = jnp.zeros((D, D), jnp.float32)
        b_sum = jnp.zeros((D,), jnp.float32)
        for j in range(5):
            wr_sum = wr_sum + wr_ref[j]
            b_sum = b_sum + b_ref[0, j]
        h = acc + jnp.dot(x_ref[0], wr_sum,
                          preferred_element_type=jnp.float32) + b_sum[None, :]
        o_ref[0] = _ln_act(h, k, g_ref, bb_ref)

    return pl.pallas_call(
        body,
        grid=(2, NBLK),
        in_specs=[
            pl.BlockSpec((5, ROWB, D),
                         lambda g, nb: (0, jnp.where(g == 0, nb, 0), 0)),
            pl.BlockSpec((2, ROWB, D),
                         lambda g, nb: (0, jnp.where(g == 1, nb, 0), 0)),
            pl.BlockSpec((1, 5, ROWB),
                         lambda g, nb: (0, 0, jnp.where(g == 0, nb, 0))),
            pl.BlockSpec((1, 2, ROWB),
                         lambda g, nb: (0, 0, jnp.where(g == 1, nb, 0))),
            pl.BlockSpec((1, ROWB, D), lambda g, nb: (g, nb, 0)),
            pl.BlockSpec((5, D, D), lambda g, nb: (g, 0, 0)),
            pl.BlockSpec((1, 5, D), lambda g, nb: (g, 0, 0)),
            pl.BlockSpec((1, D), lambda g, nb: (0, 0)),
            pl.BlockSpec((1, D), lambda g, nb: (0, 0)),
        ],
        out_specs=pl.BlockSpec((1, ROWB, D), lambda g, nb: (g, nb, 0)),
        out_shape=jax.ShapeDtypeStruct((2, N, D), jnp.float32),
    )(s_camp, s_chan, inv_camp, inv_chan, xa, wr_a, b_a, ln_g, ln_b)


def _final(xa, xb, wa, wb, b):
    def body(xa_ref, xb_ref, wa_ref, wb_ref, b_ref, o_ref):
        acc = jnp.zeros((ROWB, 1), jnp.float32)
        for t in range(2):
            acc = acc + jnp.sum(xa_ref[t] * wa_ref[t][None, :], axis=1,
                                keepdims=True)
        for t in range(7):
            acc = acc + jnp.sum(xb_ref[t] * wb_ref[t][None, :], axis=1,
                                keepdims=True)
        o_ref[...] = jax.nn.sigmoid(acc + b_ref[0, 0])

    return pl.pallas_call(
        body,
        grid=(NBLK,),
        in_specs=[
            pl.BlockSpec((2, ROWB, D), lambda nb: (0, nb, 0)),
            pl.BlockSpec((7, ROWB, D), lambda nb: (0, nb, 0)),
            pl.BlockSpec((2, D), lambda nb: (0, 0)),
            pl.BlockSpec((7, D), lambda nb: (0, 0)),
            pl.BlockSpec((1, 1), lambda nb: (0, 0)),
        ],
        out_specs=pl.BlockSpec((ROWB, 1), lambda nb: (nb, 0)),
        out_shape=jax.ShapeDtypeStruct((N, 1), jnp.float32),
    )(xa, xb, wa, wb, b)


def kernel(ei_campaign_hosted_on_platform, ei_platform_rev_hosted_on_campaign, ei_campaign_uses_channel, ei_channel_rev_uses_campaign, ei_platform_supports_channel, ei_campaign_uses_creative, ei_creative_rev_uses_campaign, ei_creative_designed_with_template, ei_campaign_associated_with_keywords, ei_keywords_rev_associated_with_campaign, ei_campaign_managed_by_network, ei_platform_optimized_for_keywords, ei_campaign_belongs_to_advertiser, ei_campaign_targeted_with_search_tag, ei_search_tag_rev_targeted_with_campaign, x_campaign, x_platform, x_channel, x_creative, x_keywords, x_search_tag, x_advertiser, x_template, x_network, p0_0_wl, p0_0_wr, p0_0_b, p0_1_wl, p0_1_wr, p0_1_b, p0_2_wl, p0_2_wr, p0_2_b, p0_3_wl, p0_3_wr, p0_3_b, p0_4_wl, p0_4_wr, p0_4_b, p0_5_wl, p0_5_wr, p0_5_b, p0_6_wl, p0_6_wr, p0_6_b, p0_7_wl, p0_7_wr, p0_7_b, p0_8_wl, p0_8_wr, p0_8_b, p0_9_wl, p0_9_wr, p0_9_b, p0_10_wl, p0_10_wr, p0_10_b, p0_11_wl, p0_11_wr, p0_11_b, p0_12_wl, p0_12_wr, p0_12_b, p0_13_wl, p0_13_wr, p0_13_b, p0_14_wl, p0_14_wr, p0_14_b, ln0_g, ln0_b, p1_0_wl, p1_0_wr, p1_0_b, p1_1_wl, p1_1_wr, p1_1_b, p1_2_wl, p1_2_wr, p1_2_b, p1_3_wl, p1_3_wr, p1_3_b, p1_4_wl, p1_4_wr, p1_4_b, p1_5_wl, p1_5_wr, p1_5_b, p1_6_wl, p1_6_wr, p1_6_b, p1_7_wl, p1_7_wr, p1_7_b, p1_8_wl, p1_8_wr, p1_8_b, p1_9_wl, p1_9_wr, p1_9_b, p1_10_wl, p1_10_wr, p1_10_b, p1_11_wl, p1_11_wr, p1_11_b, p1_12_wl, p1_12_wr, p1_12_b, p1_13_wl, p1_13_wr, p1_13_b, p1_14_wl, p1_14_wr, p1_14_b, ln1_g, ln1_b, p2_0_wl, p2_0_wr, p2_0_b, p2_1_wl, p2_1_wr, p2_1_b, p2_2_wl, p2_2_wr, p2_2_b, p2_3_wl, p2_3_wr, p2_3_b, p2_4_wl, p2_4_wr, p2_4_b, p2_5_wl, p2_5_wr, p2_5_b, p2_6_wl, p2_6_wr, p2_6_b, p2_7_wl, p2_7_wr, p2_7_b, p2_8_wl, p2_8_wr, p2_8_b, p2_9_wl, p2_9_wr, p2_9_b, p2_10_wl, p2_10_wr, p2_10_b, p2_11_wl, p2_11_wr, p2_11_b, p2_12_wl, p2_12_wr, p2_12_b, p2_13_wl, p2_13_wr, p2_13_b, p2_14_wl, p2_14_wr, p2_14_b, ln2_g, ln2_b, fc_w, fc_b):
    kw = dict(locals())
    eis = [
        ei_campaign_hosted_on_platform, ei_platform_rev_hosted_on_campaign,
        ei_campaign_uses_channel, ei_channel_rev_uses_campaign,
        ei_platform_supports_channel, ei_campaign_uses_creative,
        ei_creative_rev_uses_campaign, ei_creative_designed_with_template,
        ei_campaign_associated_with_keywords,
        ei_keywords_rev_associated_with_campaign,
        ei_campaign_managed_by_network, ei_platform_optimized_for_keywords,
        ei_campaign_belongs_to_advertiser, ei_campaign_targeted_with_search_tag,
        ei_search_tag_rev_targeted_with_campaign,
    ]
    xa = jnp.stack([x_campaign, x_channel])
    xb = jnp.stack([x_platform, x_creative, x_keywords, x_search_tag,
                    x_advertiser, x_template, x_network])

    srcs, dsts = [], []
    for slot, rel in enumerate(SLOT2REL):
        ei = eis[rel]
        s_ = jnp.concatenate([ei[0], jnp.zeros((EP - E,), jnp.int32)])
        d_ = jnp.concatenate([ei[1], jnp.full((EP - E,), N, jnp.int32)])
        half_slot = slot if slot < NSA else slot - NSA
        srcs.append(s_ + half_slot * N)
        dsts.append(d_)
    srcs_a = jnp.concatenate(srcs[:NSA])
    srcs_b = jnp.concatenate(srcs[NSA:])
    dsts_all = jnp.concatenate(dsts)
    dsts_a = jnp.concatenate(dsts[:NSA])
    dsts_b = jnp.concatenate(dsts[NSA:])
    zeros_rp = jnp.zeros((RP, D), jnp.float32)
    ones_ch = jnp.ones((CH, D), jnp.float32)

    cnt_full = _make_sc_count()(dsts_all, zeros_rp, ones_ch)
    inv_camp, inv_chan, inv_keyw, inv_sing = _compress_counts(cnt_full)

    agg_a = _make_sc_agg(CORE_A, (5, 2))
    agg_b = _make_sc_agg(CORE_B, (2, 6))

    zero_w = jnp.zeros((D, D), jnp.float32)
    zero_b = jnp.zeros((D,), jnp.float32)
    B_SLOTS = [[9], [10], [7, 8], [14], [13], [11], [12]]
    A_SLOTS = [[0, 1, 2, 3, 4], [5, 6]]
    for L in range(3):
        wl = [kw["p%d_%d_wl" % (L, r)] for r in SLOT2REL]
        wr = [kw["p%d_%d_wr" % (L, r)] for r in SLOT2REL]
        bbs = [kw["p%d_%d_b" % (L, r)] for r in SLOT2REL]
        lng = kw["ln%d_g" % L].reshape(1, D)
        lnb = kw["ln%d_b" % L].reshape(1, D)
        wr_a = jnp.stack(
            [wr[sl[j]] if j < len(sl) else zero_w
             for sl in A_SLOTS for j in range(5)])
        b_a = jnp.stack(
            [bbs[sl[j]] if j < len(sl) else zero_b
             for sl in A_SLOTS for j in range(5)]).reshape(2, 5, D)
        wr_b = jnp.stack(
            [wr[sl[j]] if j < len(sl) else zero_w
             for sl in B_SLOTS for j in range(2)])
        b_b = jnp.stack(
            [bbs[sl[j]] if j < len(sl) else zero_b
             for sl in B_SLOTS for j in range(2)]).reshape(7, 2, D)

        y_ks = _build_y(xa, xb, jnp.stack(wl[NSA:]), tuple(SRC_T[NSA:]))
        s_keyw, s_sing = agg_b(y_ks.reshape(NSB * N, D), srcs_b, dsts_b,
                               zeros_rp)
        y_cc = _build_y(xa, xb, jnp.stack(wl[:NSA]), tuple(SRC_T[:NSA]))
        s_camp, s_chan = agg_a(y_cc.reshape(NSA * N, D), srcs_a, dsts_a,
                               zeros_rp)
        xb = _combine_ks(s_keyw, s_sing, inv_keyw, inv_sing, xb, wr_b, b_b,
                         lng, lnb)
        xa = _combine_cc(s_camp, s_chan, inv_camp, inv_chan, xa, wr_a, b_a,
                         lng, lnb)

    w9 = fc_w[:, 0].reshape(NT, D)
    wa = jnp.stack([w9[0], w9[2]])
    wb = jnp.stack([w9[t] for t in TB])
    out = _final(xa, xb, wa, wb, fc_b.reshape(1, 1))
    return out[:, 0]
